# Initial kernel scaffold; baseline (speedup 1.0000x reference)
import jax
import jax.numpy as jnp
from jax.experimental import pallas as pl


def kernel(x, edge_index, edge_attr, connectivity, bc_disp, prop_I22, enc_W1, enc_b1, enc_W2, enc_b2, conv0_eW, conv0_eb, conv0_nW, conv0_nb, conv1_eW, conv1_eb, conv1_nW, conv1_nb, conv2_eW, conv2_eb, conv2_nW, conv2_nb, ef_W1, ef_b1, ef_W2, ef_b2, ef_W3, ef_b3):
    raise NotImplementedError("write your pallas kernel here")


# speedup vs baseline: 1.1100x; 1.1100x over previous
import functools

import jax
import jax.numpy as jnp
from jax.experimental import pallas as pl

N = 10000
E = 160000
H = 128
NODE_IN = 9
EDGE_DIM = 11
NPTS = 5

NP = 10240
EP = 163840
BN = 512
BE = 2048

_XI = [0.0, 0.25, 0.5, 0.75, 1.0]


def _silu(v):
    return v * jax.nn.sigmoid(v)


def _enc_body(x_ref, w1_ref, b1_ref, w2_ref, b2_ref, ws_ref, wd_ref,
              h_ref, a_ref, b_ref):
    h1 = _silu(jnp.dot(x_ref[...], w1_ref[...],
                       preferred_element_type=jnp.float32) + b1_ref[...])
    h2 = _silu(jnp.dot(h1, w2_ref[...],
                       preferred_element_type=jnp.float32) + b2_ref[...])
    h_ref[...] = h2
    a_ref[...] = jnp.dot(h2, ws_ref[...], preferred_element_type=jnp.float32)
    b_ref[...] = jnp.dot(h2, wd_ref[...], preferred_element_type=jnp.float32)


def _encode(x, w1, b1, w2, b2, ws, wd):
    grid = (NP // BN,)
    full = lambda i: (0, 0)
    return pl.pallas_call(
        _enc_body,
        grid=grid,
        in_specs=[
            pl.BlockSpec((BN, NODE_IN), lambda i: (i, 0)),
            pl.BlockSpec((NODE_IN, H), full),
            pl.BlockSpec((1, H), full),
            pl.BlockSpec((H, H), full),
            pl.BlockSpec((1, H), full),
            pl.BlockSpec((H, H), full),
            pl.BlockSpec((H, H), full),
        ],
        out_specs=[
            pl.BlockSpec((BN, H), lambda i: (i, 0)),
            pl.BlockSpec((BN, H), lambda i: (i, 0)),
            pl.BlockSpec((BN, H), lambda i: (i, 0)),
        ],
        out_shape=[jax.ShapeDtypeStruct((NP, H), jnp.float32)] * 3,
    )(x, w1, b1, w2, b2, ws, wd)


def _edge_c_body(ea_ref, wa_ref, eb_ref, c_ref):
    c_ref[...] = jnp.dot(ea_ref[...], wa_ref[...],
                         preferred_element_type=jnp.float32) + eb_ref[...]


def _edge_c(ea, wa, eb):
    return pl.pallas_call(
        _edge_c_body,
        grid=(EP // BE,),
        in_specs=[
            pl.BlockSpec((BE, EDGE_DIM), lambda i: (i, 0)),
            pl.BlockSpec((EDGE_DIM, H), lambda i: (0, 0)),
            pl.BlockSpec((1, H), lambda i: (0, 0)),
        ],
        out_specs=pl.BlockSpec((BE, H), lambda i: (i, 0)),
        out_shape=jax.ShapeDtypeStruct((EP, H), jnp.float32),
    )(ea, wa, eb)


def _edge_m_body(ag_ref, bg_ref, c_ref, m_ref):
    m_ref[...] = _silu(ag_ref[...] + bg_ref[...] + c_ref[...])


def _edge_m(ag, bg, c):
    return pl.pallas_call(
        _edge_m_body,
        grid=(EP // BE,),
        in_specs=[pl.BlockSpec((BE, H), lambda i: (i, 0))] * 3,
        out_specs=pl.BlockSpec((BE, H), lambda i: (i, 0)),
        out_shape=jax.ShapeDtypeStruct((EP, H), jnp.float32),
    )(ag, bg, c)


def _node_upd_body(h_ref, agg_ref, w1_ref, w2_ref, nb_ref, ws_ref, wd_ref,
                   h_ref_o, a_ref, b_ref):
    hn = _silu(jnp.dot(h_ref[...], w1_ref[...],
                       preferred_element_type=jnp.float32)
               + jnp.dot(agg_ref[...], w2_ref[...],
                         preferred_element_type=jnp.float32)
               + nb_ref[...])
    h_ref_o[...] = hn
    a_ref[...] = jnp.dot(hn, ws_ref[...], preferred_element_type=jnp.float32)
    b_ref[...] = jnp.dot(hn, wd_ref[...], preferred_element_type=jnp.float32)


def _node_update(h, agg, nw1, nw2, nb, ws, wd):
    full = lambda i: (0, 0)
    return pl.pallas_call(
        _node_upd_body,
        grid=(NP // BN,),
        in_specs=[
            pl.BlockSpec((BN, H), lambda i: (i, 0)),
            pl.BlockSpec((BN, H), lambda i: (i, 0)),
            pl.BlockSpec((H, H), full),
            pl.BlockSpec((H, H), full),
            pl.BlockSpec((1, H), full),
            pl.BlockSpec((H, H), full),
            pl.BlockSpec((H, H), full),
        ],
        out_specs=[pl.BlockSpec((BN, H), lambda i: (i, 0))] * 3,
        out_shape=[jax.ShapeDtypeStruct((NP, H), jnp.float32)] * 3,
    )(h, agg, nw1, nw2, nb, ws, wd)


def _node_final_body(h_ref, agg_ref, w1_ref, w2_ref, nb_ref,
                     wi_ref, wj_ref, bc_ref,
                     h_ref_o, t1_ref, t2_ref):
    hn = _silu(jnp.dot(h_ref[...], w1_ref[...],
                       preferred_element_type=jnp.float32)
               + jnp.dot(agg_ref[...], w2_ref[...],
                         preferred_element_type=jnp.float32)
               + nb_ref[...])
    h_ref_o[...] = hn
    bc = bc_ref[...]
    xi = 0.25 * jax.lax.broadcasted_iota(
        jnp.int32, (1, NPTS), 1).astype(jnp.float32)
    mi = 1.0 - bc * (1.0 - xi)
    mj = 1.0 - bc * xi
    ha = jnp.dot(hn, wi_ref[...], preferred_element_type=jnp.float32)
    hb = jnp.dot(hn, wj_ref[...], preferred_element_type=jnp.float32)
    zpad = jnp.zeros((hn.shape[0], 80 - 64 - NPTS), jnp.float32)
    t1_ref[...] = jnp.concatenate([ha, mi, zpad], axis=1)
    t2_ref[...] = jnp.concatenate([hb, mj, zpad], axis=1)


def _node_final(h, agg, nw1, nw2, nb, wi, wj, bc):
    full = lambda i: (0, 0)
    return pl.pallas_call(
        _node_final_body,
        grid=(NP // BN,),
        in_specs=[
            pl.BlockSpec((BN, H), lambda i: (i, 0)),
            pl.BlockSpec((BN, H), lambda i: (i, 0)),
            pl.BlockSpec((H, H), full),
            pl.BlockSpec((H, H), full),
            pl.BlockSpec((1, H), full),
            pl.BlockSpec((H, 64), full),
            pl.BlockSpec((H, 64), full),
            pl.BlockSpec((BN, 1), lambda i: (i, 0)),
        ],
        out_specs=[
            pl.BlockSpec((BN, H), lambda i: (i, 0)),
            pl.BlockSpec((BN, 80), lambda i: (i, 0)),
            pl.BlockSpec((BN, 80), lambda i: (i, 0)),
        ],
        out_shape=[
            jax.ShapeDtypeStruct((NP, H), jnp.float32),
            jax.ShapeDtypeStruct((NP, 80), jnp.float32),
            jax.ShapeDtypeStruct((NP, 80), jnp.float32),
        ],
    )(h, agg, nw1, nw2, nb, wi, wj, bc)


def _field_body(g1_ref, g2_ref, w0_ref, b1_ref, w2_ref, b2_ref, w3_ref,
                b3_ref, f_ref):
    g1 = g1_ref[...]
    g2 = g2_ref[...]
    base = g1[:, :64] + g2[:, :64]
    w0 = w0_ref[...]
    for k in range(NPTS):
        z = _silu(base + _XI[k] * w0 + b1_ref[...])
        z = _silu(jnp.dot(z, w2_ref[...],
                          preferred_element_type=jnp.float32) + b2_ref[...])
        f = jnp.dot(z, w3_ref[...], preferred_element_type=jnp.float32) \
            + b3_ref[...]
        mask = (g1[:, 64 + k] * g2[:, 64 + k])[:, None]
        f_ref[:, 3 * k:3 * k + 3] = f * mask


def _field(g1, g2, w0, b1, w2, b2, w3, b3):
    full = lambda i: (0, 0)
    return pl.pallas_call(
        _field_body,
        grid=(EP // BE,),
        in_specs=[
            pl.BlockSpec((BE, 80), lambda i: (i, 0)),
            pl.BlockSpec((BE, 80), lambda i: (i, 0)),
            pl.BlockSpec((1, 64), full),
            pl.BlockSpec((1, 64), full),
            pl.BlockSpec((64, 64), full),
            pl.BlockSpec((1, 64), full),
            pl.BlockSpec((64, 3), full),
            pl.BlockSpec((1, 3), full),
        ],
        out_specs=pl.BlockSpec((BE, NPTS * 3), lambda i: (i, 0)),
        out_shape=jax.ShapeDtypeStruct((EP, NPTS * 3), jnp.float32),
    )(g1, g2, w0, b1, w2, b2, w3, b3)


def kernel(x, edge_index, edge_attr, connectivity, bc_disp, prop_I22,
           enc_W1, enc_b1, enc_W2, enc_b2,
           conv0_eW, conv0_eb, conv0_nW, conv0_nb,
           conv1_eW, conv1_eb, conv1_nW, conv1_nb,
           conv2_eW, conv2_eb, conv2_nW, conv2_nb,
           ef_W1, ef_b1, ef_W2, ef_b2, ef_W3, ef_b3):
    convs = [(conv0_eW, conv0_eb, conv0_nW, conv0_nb),
             (conv1_eW, conv1_eb, conv1_nW, conv1_nb),
             (conv2_eW, conv2_eb, conv2_nW, conv2_nb)]

    x_p = jnp.pad(x, ((0, NP - N), (0, 0)))
    bc_p = jnp.pad(bc_disp, ((0, NP - N), (0, 0)))
    src = jnp.pad(edge_index[0], (0, EP - E))
    dst = jnp.pad(edge_index[1], (0, EP - E), constant_values=N)
    ea_p = jnp.pad(edge_attr, ((0, EP - E), (0, 0)))
    n1 = jnp.pad(connectivity[:, 0], (0, EP - E))
    n2 = jnp.pad(connectivity[:, 1], (0, EP - E))

    r2 = lambda v: v.reshape(1, -1)

    h, a, b = _encode(x_p, enc_W1, r2(enc_b1), enc_W2, r2(enc_b2),
                      convs[0][0][:H], convs[0][0][H:2 * H])

    for li in range(3):
        eW, eb, nW, nb = convs[li]
        c = _edge_c(ea_p, eW[2 * H:], r2(eb))
        ag = jnp.take(a, src, axis=0)
        bg = jnp.take(b, dst, axis=0)
        m = _edge_m(ag, bg, c)
        agg = jax.ops.segment_sum(m, dst, num_segments=NP)
        if li < 2:
            nws, nwd = convs[li + 1][0][:H], convs[li + 1][0][H:2 * H]
            h, a, b = _node_update(h, agg, nW[:H], nW[H:], r2(nb), nws, nwd)
        else:
            h, t1, t2 = _node_final(h, agg, nW[:H], nW[H:], r2(nb),
                                    ef_W1[1:1 + H], ef_W1[1 + H:], bc_p)

    g1 = jnp.take(t1, n1, axis=0)
    g2 = jnp.take(t2, n2, axis=0)
    f = _field(g1, g2, r2(ef_W1[0]), r2(ef_b1), ef_W2, r2(ef_b2),
               ef_W3, r2(ef_b3))

    fields = f[:E].reshape(E, NPTS, 3)
    xi = jnp.broadcast_to(
        jnp.asarray(_XI, jnp.float32)[None, :, None], (E, NPTS, 1))
    return h[:N], xi, fields, prop_I22


# speedup vs baseline: 3.4297x; 3.0899x over previous
import functools

import jax
import jax.numpy as jnp
from jax import lax
from jax.experimental import pallas as pl
from jax.experimental.pallas import tpu as pltpu
from jax.experimental.pallas import tpu_sc as plsc

N = 10000
E = 160000
H = 128
NODE_IN = 9
EDGE_DIM = 11
NPTS = 5

NP = 10240
EP = 163840
BN = 512
BE = 2048

_XI = [0.0, 0.25, 0.5, 0.75, 1.0]

NC = 2
NS = 16
TILES = NC * NS
PT = EP // TILES
CH = 128
CCH = 80
NCCHUNK = PT // CCH
NCHUNK = PT // CH
RPS = NP // NS

_SC_MESH = plsc.VectorSubcoreMesh(core_axis_name="c", subcore_axis_name="s")


def _silu(v):
    return v * jax.nn.sigmoid(v)


def _conv_sc(a, b, c, src, dst):
    @functools.partial(
        pl.kernel,
        out_type=jax.ShapeDtypeStruct((NC, NP, H), jnp.float32),
        mesh=_SC_MESH,
        scratch_types=[
            pltpu.VMEM((CCH,), jnp.int32),
            pltpu.VMEM((CCH,), jnp.int32),
            pltpu.VMEM((CCH, H), jnp.float32),
            pltpu.VMEM((CCH, H), jnp.float32),
            pltpu.VMEM((CCH, H), jnp.float32),
            pltpu.VMEM_SHARED((NP, H), jnp.float32),
            pltpu.SemaphoreType.DMA,
            pltpu.SemaphoreType.DMA,
        ],
    )
    def k(a_hbm, b_hbm, c_hbm, src_hbm, dst_hbm, out_hbm,
          si_v, di_v, ar_v, br_v, cr_v, acc, sem1, sem2):
        cid = lax.axis_index("c")
        sid = lax.axis_index("s")
        wid = sid * NC + cid
        zrow = sid * RPS

        @pl.loop(0, CCH)
        def _(r):
            @pl.loop(0, H, step=16)
            def _(cc):
                ar_v[r, pl.ds(cc, 16)] = jnp.zeros((16,), jnp.float32)

        @pl.loop(0, RPS, step=CCH)
        def _(rr):
            pltpu.sync_copy(ar_v, acc.at[pl.ds(zrow + rr, CCH)])

        plsc.subcore_barrier()

        base = wid * PT

        @pl.loop(0, NCCHUNK)
        def _(ci):
            off = base + ci * CCH
            pltpu.sync_copy(src_hbm.at[pl.ds(off, CCH)], si_v)
            pltpu.sync_copy(dst_hbm.at[pl.ds(off, CCH)], di_v)
            cp1 = pltpu.async_copy(a_hbm.at[si_v], ar_v, sem1)
            cp2 = pltpu.async_copy(b_hbm.at[di_v], br_v, sem2)
            pltpu.sync_copy(c_hbm.at[pl.ds(off, CCH)], cr_v)
            cp1.wait()
            cp2.wait()

            @pl.loop(0, CCH)
            def _(r):
                @pl.loop(0, H, step=16)
                def _(cc):
                    v = (ar_v[r, pl.ds(cc, 16)] + br_v[r, pl.ds(cc, 16)]
                         + cr_v[r, pl.ds(cc, 16)])
                    ar_v[r, pl.ds(cc, 16)] = v / (1.0 + jnp.exp(-v))

            pltpu.sync_copy(ar_v, acc.at[di_v], add=True)

        plsc.subcore_barrier()
        pltpu.sync_copy(acc.at[pl.ds(zrow, RPS)],
                        out_hbm.at[cid, pl.ds(zrow, RPS)])

    return k(a, b, c, src, dst)


def _field_sc(t1, t2, n1, n2):
    @functools.partial(
        pl.kernel,
        out_type=jax.ShapeDtypeStruct((EP, 128), jnp.float32),
        mesh=_SC_MESH,
        scratch_types=[
            pltpu.VMEM((CH,), jnp.int32),
            pltpu.VMEM((CH,), jnp.int32),
            pltpu.VMEM((CH, 128), jnp.float32),
            pltpu.VMEM((CH, 128), jnp.float32),
            pltpu.SemaphoreType.DMA,
            pltpu.SemaphoreType.DMA,
        ],
    )
    def k(t1_hbm, t2_hbm, n1_hbm, n2_hbm, out_hbm,
          i1_v, i2_v, g1_v, g2_v, sem1, sem2):
        cid = lax.axis_index("c")
        sid = lax.axis_index("s")
        wid = sid * NC + cid
        base = wid * PT

        @pl.loop(0, NCHUNK)
        def _(ci):
            off = base + ci * CH
            pltpu.sync_copy(n1_hbm.at[pl.ds(off, CH)], i1_v)
            pltpu.sync_copy(n2_hbm.at[pl.ds(off, CH)], i2_v)
            cp1 = pltpu.async_copy(t1_hbm.at[i1_v], g1_v, sem1)
            cp2 = pltpu.async_copy(t2_hbm.at[i2_v], g2_v, sem2)
            cp1.wait()
            cp2.wait()

            @pl.loop(0, CH)
            def _(r):
                @pl.loop(0, 64, step=16)
                def _(cc):
                    g1_v[r, pl.ds(cc, 16)] = (g1_v[r, pl.ds(cc, 16)]
                                              + g2_v[r, pl.ds(cc, 16)])
                g1_v[r, pl.ds(64, 16)] = (g1_v[r, pl.ds(64, 16)]
                                          * g2_v[r, pl.ds(64, 16)])

            pltpu.sync_copy(g1_v, out_hbm.at[pl.ds(off, CH)])

    return k(t1, t2, n1, n2)


def _enc_body(x_ref, w1_ref, b1_ref, w2_ref, b2_ref, ws_ref, wd_ref,
              h_ref, a_ref, b_ref):
    h1 = _silu(jnp.dot(x_ref[...], w1_ref[...],
                       preferred_element_type=jnp.float32) + b1_ref[...])
    h2 = _silu(jnp.dot(h1, w2_ref[...],
                       preferred_element_type=jnp.float32) + b2_ref[...])
    h_ref[...] = h2
    a_ref[...] = jnp.dot(h2, ws_ref[...], preferred_element_type=jnp.float32)
    b_ref[...] = jnp.dot(h2, wd_ref[...], preferred_element_type=jnp.float32)


def _encode(x, w1, b1, w2, b2, ws, wd):
    grid = (NP // BN,)
    full = lambda i: (0, 0)
    return pl.pallas_call(
        _enc_body,
        grid=grid,
        in_specs=[
            pl.BlockSpec((BN, NODE_IN), lambda i: (i, 0)),
            pl.BlockSpec((NODE_IN, H), full),
            pl.BlockSpec((1, H), full),
            pl.BlockSpec((H, H), full),
            pl.BlockSpec((1, H), full),
            pl.BlockSpec((H, H), full),
            pl.BlockSpec((H, H), full),
        ],
        out_specs=[
            pl.BlockSpec((BN, H), lambda i: (i, 0)),
            pl.BlockSpec((BN, H), lambda i: (i, 0)),
            pl.BlockSpec((BN, H), lambda i: (i, 0)),
        ],
        out_shape=[jax.ShapeDtypeStruct((NP, H), jnp.float32)] * 3,
    )(x, w1, b1, w2, b2, ws, wd)


def _edge_c_body(ea_ref, wa_ref, eb_ref, c_ref):
    c_ref[...] = jnp.dot(ea_ref[...], wa_ref[...],
                         preferred_element_type=jnp.float32) + eb_ref[...]


def _edge_c(ea, wa, eb):
    return pl.pallas_call(
        _edge_c_body,
        grid=(EP // BE,),
        in_specs=[
            pl.BlockSpec((BE, EDGE_DIM), lambda i: (i, 0)),
            pl.BlockSpec((EDGE_DIM, H), lambda i: (0, 0)),
            pl.BlockSpec((1, H), lambda i: (0, 0)),
        ],
        out_specs=pl.BlockSpec((BE, H), lambda i: (i, 0)),
        out_shape=jax.ShapeDtypeStruct((EP, H), jnp.float32),
    )(ea, wa, eb)


def _node_upd_body(h_ref, agg0_ref, agg1_ref, w1_ref, w2_ref, nb_ref,
                   ws_ref, wd_ref, h_ref_o, a_ref, b_ref):
    agg = agg0_ref[0] + agg1_ref[0]
    hn = _silu(jnp.dot(h_ref[...], w1_ref[...],
                       preferred_element_type=jnp.float32)
               + jnp.dot(agg, w2_ref[...],
                         preferred_element_type=jnp.float32)
               + nb_ref[...])
    h_ref_o[...] = hn
    a_ref[...] = jnp.dot(hn, ws_ref[...], preferred_element_type=jnp.float32)
    b_ref[...] = jnp.dot(hn, wd_ref[...], preferred_element_type=jnp.float32)


def _node_update(h, aggp, nw1, nw2, nb, ws, wd):
    full = lambda i: (0, 0)
    return pl.pallas_call(
        _node_upd_body,
        grid=(NP // BN,),
        in_specs=[
            pl.BlockSpec((BN, H), lambda i: (i, 0)),
            pl.BlockSpec((1, BN, H), lambda i: (0, i, 0)),
            pl.BlockSpec((1, BN, H), lambda i: (1, i, 0)),
            pl.BlockSpec((H, H), full),
            pl.BlockSpec((H, H), full),
            pl.BlockSpec((1, H), full),
            pl.BlockSpec((H, H), full),
            pl.BlockSpec((H, H), full),
        ],
        out_specs=[pl.BlockSpec((BN, H), lambda i: (i, 0))] * 3,
        out_shape=[jax.ShapeDtypeStruct((NP, H), jnp.float32)] * 3,
    )(h, aggp, aggp, nw1, nw2, nb, ws, wd)


def _node_final_body(h_ref, agg0_ref, agg1_ref, w1_ref, w2_ref, nb_ref,
                     wi_ref, wj_ref, bc_ref,
                     h_ref_o, t1_ref, t2_ref):
    agg = agg0_ref[0] + agg1_ref[0]
    hn = _silu(jnp.dot(h_ref[...], w1_ref[...],
                       preferred_element_type=jnp.float32)
               + jnp.dot(agg, w2_ref[...],
                         preferred_element_type=jnp.float32)
               + nb_ref[...])
    h_ref_o[...] = hn
    bc = bc_ref[...]
    xi = 0.25 * jax.lax.broadcasted_iota(
        jnp.int32, (1, NPTS), 1).astype(jnp.float32)
    mi = 1.0 - bc * (1.0 - xi)
    mj = 1.0 - bc * xi
    ha = jnp.dot(hn, wi_ref[...], preferred_element_type=jnp.float32)
    hb = jnp.dot(hn, wj_ref[...], preferred_element_type=jnp.float32)
    zpad = jnp.zeros((hn.shape[0], 128 - 64 - NPTS), jnp.float32)
    t1_ref[...] = jnp.concatenate([ha, mi, zpad], axis=1)
    t2_ref[...] = jnp.concatenate([hb, mj, zpad], axis=1)


def _node_final(h, aggp, nw1, nw2, nb, wi, wj, bc):
    full = lambda i: (0, 0)
    return pl.pallas_call(
        _node_final_body,
        grid=(NP // BN,),
        in_specs=[
            pl.BlockSpec((BN, H), lambda i: (i, 0)),
            pl.BlockSpec((1, BN, H), lambda i: (0, i, 0)),
            pl.BlockSpec((1, BN, H), lambda i: (1, i, 0)),
            pl.BlockSpec((H, H), full),
            pl.BlockSpec((H, H), full),
            pl.BlockSpec((1, H), full),
            pl.BlockSpec((H, 64), full),
            pl.BlockSpec((H, 64), full),
            pl.BlockSpec((BN, 1), lambda i: (i, 0)),
        ],
        out_specs=[
            pl.BlockSpec((BN, H), lambda i: (i, 0)),
            pl.BlockSpec((BN, 128), lambda i: (i, 0)),
            pl.BlockSpec((BN, 128), lambda i: (i, 0)),
        ],
        out_shape=[
            jax.ShapeDtypeStruct((NP, H), jnp.float32),
            jax.ShapeDtypeStruct((NP, 128), jnp.float32),
            jax.ShapeDtypeStruct((NP, 128), jnp.float32),
        ],
    )(h, aggp, aggp, nw1, nw2, nb, wi, wj, bc)


def _field_body(g_ref, w0_ref, b1_ref, w2_ref, b2_ref, w3_ref,
                b3_ref, f_ref):
    g = g_ref[...]
    base = g[:, :64]
    w0 = w0_ref[...]
    for k in range(NPTS):
        z = _silu(base + _XI[k] * w0 + b1_ref[...])
        z = _silu(jnp.dot(z, w2_ref[...],
                          preferred_element_type=jnp.float32) + b2_ref[...])
        f = jnp.dot(z, w3_ref[...], preferred_element_type=jnp.float32) \
            + b3_ref[...]
        mask = g[:, 64 + k][:, None]
        f_ref[:, 3 * k:3 * k + 3] = f * mask


def _field(g, w0, b1, w2, b2, w3, b3):
    full = lambda i: (0, 0)
    return pl.pallas_call(
        _field_body,
        grid=(EP // BE,),
        in_specs=[
            pl.BlockSpec((BE, 128), lambda i: (i, 0)),
            pl.BlockSpec((1, 64), full),
            pl.BlockSpec((1, 64), full),
            pl.BlockSpec((64, 64), full),
            pl.BlockSpec((1, 64), full),
            pl.BlockSpec((64, 3), full),
            pl.BlockSpec((1, 3), full),
        ],
        out_specs=pl.BlockSpec((BE, NPTS * 3), lambda i: (i, 0)),
        out_shape=jax.ShapeDtypeStruct((EP, NPTS * 3), jnp.float32),
    )(g, w0, b1, w2, b2, w3, b3)


def kernel(x, edge_index, edge_attr, connectivity, bc_disp, prop_I22,
           enc_W1, enc_b1, enc_W2, enc_b2,
           conv0_eW, conv0_eb, conv0_nW, conv0_nb,
           conv1_eW, conv1_eb, conv1_nW, conv1_nb,
           conv2_eW, conv2_eb, conv2_nW, conv2_nb,
           ef_W1, ef_b1, ef_W2, ef_b2, ef_W3, ef_b3):
    convs = [(conv0_eW, conv0_eb, conv0_nW, conv0_nb),
             (conv1_eW, conv1_eb, conv1_nW, conv1_nb),
             (conv2_eW, conv2_eb, conv2_nW, conv2_nb)]

    x_p = jnp.pad(x, ((0, NP - N), (0, 0)))
    bc_p = jnp.pad(bc_disp, ((0, NP - N), (0, 0)))
    src = jnp.pad(edge_index[0], (0, EP - E))
    dst = jnp.pad(edge_index[1], (0, EP - E), constant_values=N)
    ea_p = jnp.pad(edge_attr, ((0, EP - E), (0, 0)))
    n1 = jnp.pad(connectivity[:, 0], (0, EP - E))
    n2 = jnp.pad(connectivity[:, 1], (0, EP - E))

    r2 = lambda v: v.reshape(1, -1)

    h, a, b = _encode(x_p, enc_W1, r2(enc_b1), enc_W2, r2(enc_b2),
                      convs[0][0][:H], convs[0][0][H:2 * H])

    for li in range(3):
        eW, eb, nW, nb = convs[li]
        c = _edge_c(ea_p, eW[2 * H:], r2(eb))
        aggp = _conv_sc(a, b, c, src, dst)
        if li < 2:
            nws, nwd = convs[li + 1][0][:H], convs[li + 1][0][H:2 * H]
            h, a, b = _node_update(h, aggp, nW[:H], nW[H:], r2(nb), nws, nwd)
        else:
            h, t1, t2 = _node_final(h, aggp, nW[:H], nW[H:], r2(nb),
                                    ef_W1[1:1 + H], ef_W1[1 + H:], bc_p)

    g = _field_sc(t1, t2, n1, n2)
    f = _field(g, r2(ef_W1[0]), r2(ef_b1), ef_W2, r2(ef_b2),
               ef_W3, r2(ef_b3))

    fields = f[:E].reshape(E, NPTS, 3)
    xi = jnp.broadcast_to(
        jnp.asarray(_XI, jnp.float32)[None, :, None], (E, NPTS, 1))
    return h[:N], xi, fields, prop_I22


# speedup vs baseline: 4.3102x; 1.2567x over previous
import functools

import jax
import jax.numpy as jnp
from jax import lax
from jax.experimental import pallas as pl
from jax.experimental.pallas import tpu as pltpu
from jax.experimental.pallas import tpu_sc as plsc

N = 10000
E = 160000
H = 128
NODE_IN = 9
EDGE_DIM = 11
NPTS = 5

NP = 10240
EP = 163840
BN = 512
BE = 2048

_XI = [0.0, 0.25, 0.5, 0.75, 1.0]

NC = 2
NS = 16
TILES = NC * NS
PT = EP // TILES
CCH = 64
NCCHUNK = PT // CCH
NPA = 10112
RPSA = NPA // NS
CHF = 128
NF = PT // CHF

_SC_MESH = plsc.VectorSubcoreMesh(core_axis_name="c", subcore_axis_name="s")


def _silu(v):
    return v * jax.nn.sigmoid(v)


def _conv_sc(a, b, c, i2):
    @functools.partial(
        pl.kernel,
        out_type=jax.ShapeDtypeStruct((NC, NP, H), jnp.float32),
        mesh=_SC_MESH,
        scratch_types=[
            pltpu.VMEM((2, CCH), jnp.int32),
            pltpu.VMEM((2, CCH), jnp.int32),
            pltpu.VMEM((CCH, H), jnp.float32),
            pltpu.VMEM((CCH, H), jnp.float32),
            pltpu.VMEM((CCH, H), jnp.float32),
            pltpu.VMEM((CCH, H), jnp.float32),
            pltpu.VMEM((CCH, H), jnp.float32),
            pltpu.VMEM((CCH, H), jnp.float32),
            pltpu.VMEM_SHARED((NPA, H), jnp.float32),
            pltpu.SemaphoreType.DMA,
            pltpu.SemaphoreType.DMA,
            pltpu.SemaphoreType.DMA,
            pltpu.SemaphoreType.DMA,
            pltpu.SemaphoreType.DMA,
            pltpu.SemaphoreType.DMA,
        ],
    )
    def k(a_hbm, b_hbm, c_hbm, i2_hbm, out_hbm,
          ib0, ib1, ar0, ar1, br0, br1, cr0, cr1, acc,
          sa0, sa1, sb0, sb1, sc0, sc1):
        cid = lax.axis_index("c")
        sid = lax.axis_index("s")
        wid = sid * NC + cid
        zrow = sid * RPSA
        base = wid * PT

        @pl.loop(0, CCH)
        def _(r):
            @pl.loop(0, H, step=16)
            def _(cc):
                ar0[r, pl.ds(cc, 16)] = jnp.zeros((16,), jnp.float32)

        @pl.loop(0, RPSA - CCH, step=CCH)
        def _(rr):
            pltpu.sync_copy(ar0, acc.at[pl.ds(zrow + rr, CCH)])
        pltpu.sync_copy(ar0.at[pl.ds(0, RPSA % CCH)],
                        acc.at[pl.ds(zrow + RPSA - RPSA % CCH, RPSA % CCH)])

        plsc.subcore_barrier()

        bufs = ((ib0, ar0, br0, cr0, sa0, sb0, sc0),
                (ib1, ar1, br1, cr1, sa1, sb1, sc1))

        def issue(ci, ib, ar, br, cr, sa, sb, sc):
            pltpu.sync_copy(i2_hbm.at[wid, ci], ib)
            pltpu.async_copy(a_hbm.at[ib.at[0]], ar, sa)
            pltpu.async_copy(b_hbm.at[ib.at[1]], br, sb)
            pltpu.async_copy(c_hbm.at[pl.ds(base + ci * CCH, CCH)], cr, sc)

        def wait(ib, ar, br, cr, sa, sb, sc):
            pltpu.make_async_copy(a_hbm.at[ib.at[0]], ar, sa).wait()
            pltpu.make_async_copy(b_hbm.at[ib.at[1]], br, sb).wait()
            pltpu.make_async_copy(c_hbm.at[pl.ds(base, CCH)], cr, sc).wait()

        issue(0, *bufs[0])
        issue(1, *bufs[1])

        @pl.loop(0, NCCHUNK, step=2)
        def _(g):
            for p in range(2):
                ib, ar, br, cr, sa, sb, sc = bufs[p]
                ci = g + p
                wait(*bufs[p])

                @pl.loop(0, CCH)
                def _(r):
                    @pl.loop(0, H, step=16)
                    def _(cc):
                        v = (ar[r, pl.ds(cc, 16)] + br[r, pl.ds(cc, 16)]
                             + cr[r, pl.ds(cc, 16)])
                        ar[r, pl.ds(cc, 16)] = v / (1.0 + jnp.exp(-v))

                pltpu.sync_copy(ar, acc.at[ib.at[1]], add=True)

                @pl.when(ci + 2 < NCCHUNK)
                def _():
                    issue(ci + 2, *bufs[p])

        plsc.subcore_barrier()
        pltpu.sync_copy(acc.at[pl.ds(zrow, RPSA)],
                        out_hbm.at[cid, pl.ds(zrow, RPSA)])

    return k(a, b, c, i2)


def _field_sc(t1, t2, i2f):
    @functools.partial(
        pl.kernel,
        out_type=jax.ShapeDtypeStruct((EP, 128), jnp.float32),
        mesh=_SC_MESH,
        scratch_types=[
            pltpu.VMEM((2, CHF), jnp.int32),
            pltpu.VMEM((2, CHF), jnp.int32),
            pltpu.VMEM((CHF, 128), jnp.float32),
            pltpu.VMEM((CHF, 128), jnp.float32),
            pltpu.VMEM((CHF, 128), jnp.float32),
            pltpu.VMEM((CHF, 128), jnp.float32),
            pltpu.VMEM((CHF, 128), jnp.float32),
            pltpu.VMEM((CHF, 128), jnp.float32),
            pltpu.SemaphoreType.DMA,
            pltpu.SemaphoreType.DMA,
            pltpu.SemaphoreType.DMA,
            pltpu.SemaphoreType.DMA,
            pltpu.SemaphoreType.DMA,
            pltpu.SemaphoreType.DMA,
        ],
    )
    def k(t1_hbm, t2_hbm, i2_hbm, out_hbm,
          ib0, ib1, g10, g11, g20, g21, ob0, ob1,
          s10, s11, s20, s21, so0, so1):
        cid = lax.axis_index("c")
        sid = lax.axis_index("s")
        wid = sid * NC + cid
        base = wid * PT

        bufs = ((ib0, g10, g20, ob0, s10, s20, so0),
                (ib1, g11, g21, ob1, s11, s21, so1))

        def issue(ci, ib, g1, g2, ob, s1, s2, so):
            pltpu.sync_copy(i2_hbm.at[wid, ci], ib)
            pltpu.async_copy(t1_hbm.at[ib.at[0]], g1, s1)
            pltpu.async_copy(t2_hbm.at[ib.at[1]], g2, s2)

        issue(0, *bufs[0])
        issue(1, *bufs[1])

        @pl.loop(0, NF, step=2)
        def _(g):
            for p in range(2):
                ib, g1, g2, ob, s1, s2, so = bufs[p]
                ci = g + p
                pltpu.make_async_copy(t1_hbm.at[ib.at[0]], g1, s1).wait()
                pltpu.make_async_copy(t2_hbm.at[ib.at[1]], g2, s2).wait()

                @pl.when(ci >= 2)
                def _():
                    pltpu.make_async_copy(
                        ob, out_hbm.at[pl.ds(base, CHF)], so).wait()

                @pl.loop(0, CHF)
                def _(r):
                    @pl.loop(0, 64, step=16)
                    def _(cc):
                        ob[r, pl.ds(cc, 16)] = (g1[r, pl.ds(cc, 16)]
                                                + g2[r, pl.ds(cc, 16)])
                    ob[r, pl.ds(64, 16)] = (g1[r, pl.ds(64, 16)]
                                            * g2[r, pl.ds(64, 16)])

                pltpu.async_copy(ob, out_hbm.at[pl.ds(base + ci * CHF, CHF)],
                                 so)

                @pl.when(ci + 2 < NF)
                def _():
                    issue(ci + 2, *bufs[p])

        pltpu.make_async_copy(ob0, out_hbm.at[pl.ds(base, CHF)], so0).wait()
        pltpu.make_async_copy(ob1, out_hbm.at[pl.ds(base, CHF)], so1).wait()

    return k(t1, t2, i2f)


def _enc_body(x_ref, w1_ref, b1_ref, w2_ref, b2_ref, ws_ref, wd_ref,
              h_ref, a_ref, b_ref):
    h1 = _silu(jnp.dot(x_ref[...], w1_ref[...],
                       preferred_element_type=jnp.float32) + b1_ref[...])
    h2 = _silu(jnp.dot(h1, w2_ref[...],
                       preferred_element_type=jnp.float32) + b2_ref[...])
    h_ref[...] = h2
    a_ref[...] = jnp.dot(h2, ws_ref[...], preferred_element_type=jnp.float32)
    b_ref[...] = jnp.dot(h2, wd_ref[...], preferred_element_type=jnp.float32)


def _encode(x, w1, b1, w2, b2, ws, wd):
    grid = (NP // BN,)
    full = lambda i: (0, 0)
    return pl.pallas_call(
        _enc_body,
        grid=grid,
        in_specs=[
            pl.BlockSpec((BN, NODE_IN), lambda i: (i, 0)),
            pl.BlockSpec((NODE_IN, H), full),
            pl.BlockSpec((1, H), full),
            pl.BlockSpec((H, H), full),
            pl.BlockSpec((1, H), full),
            pl.BlockSpec((H, H), full),
            pl.BlockSpec((H, H), full),
        ],
        out_specs=[
            pl.BlockSpec((BN, H), lambda i: (i, 0)),
            pl.BlockSpec((BN, H), lambda i: (i, 0)),
            pl.BlockSpec((BN, H), lambda i: (i, 0)),
        ],
        out_shape=[jax.ShapeDtypeStruct((NP, H), jnp.float32)] * 3,
    )(x, w1, b1, w2, b2, ws, wd)


def _edge_c_body(ea_ref, wa_ref, eb_ref, c_ref):
    c_ref[...] = jnp.dot(ea_ref[...], wa_ref[...],
                         preferred_element_type=jnp.float32) + eb_ref[...]


def _edge_c(ea, wa, eb):
    return pl.pallas_call(
        _edge_c_body,
        grid=(EP // BE,),
        in_specs=[
            pl.BlockSpec((BE, EDGE_DIM), lambda i: (i, 0)),
            pl.BlockSpec((EDGE_DIM, H), lambda i: (0, 0)),
            pl.BlockSpec((1, H), lambda i: (0, 0)),
        ],
        out_specs=pl.BlockSpec((BE, H), lambda i: (i, 0)),
        out_shape=jax.ShapeDtypeStruct((EP, H), jnp.float32),
    )(ea, wa, eb)


def _node_upd_body(h_ref, agg0_ref, agg1_ref, w1_ref, w2_ref, nb_ref,
                   ws_ref, wd_ref, h_ref_o, a_ref, b_ref):
    agg = agg0_ref[0] + agg1_ref[0]
    hn = _silu(jnp.dot(h_ref[...], w1_ref[...],
                       preferred_element_type=jnp.float32)
               + jnp.dot(agg, w2_ref[...],
                         preferred_element_type=jnp.float32)
               + nb_ref[...])
    h_ref_o[...] = hn
    a_ref[...] = jnp.dot(hn, ws_ref[...], preferred_element_type=jnp.float32)
    b_ref[...] = jnp.dot(hn, wd_ref[...], preferred_element_type=jnp.float32)


def _node_update(h, aggp, nw1, nw2, nb, ws, wd):
    full = lambda i: (0, 0)
    return pl.pallas_call(
        _node_upd_body,
        grid=(NP // BN,),
        in_specs=[
            pl.BlockSpec((BN, H), lambda i: (i, 0)),
            pl.BlockSpec((1, BN, H), lambda i: (0, i, 0)),
            pl.BlockSpec((1, BN, H), lambda i: (1, i, 0)),
            pl.BlockSpec((H, H), full),
            pl.BlockSpec((H, H), full),
            pl.BlockSpec((1, H), full),
            pl.BlockSpec((H, H), full),
            pl.BlockSpec((H, H), full),
        ],
        out_specs=[pl.BlockSpec((BN, H), lambda i: (i, 0))] * 3,
        out_shape=[jax.ShapeDtypeStruct((NP, H), jnp.float32)] * 3,
    )(h, aggp, aggp, nw1, nw2, nb, ws, wd)


def _node_final_body(h_ref, agg0_ref, agg1_ref, w1_ref, w2_ref, nb_ref,
                     wi_ref, wj_ref, bc_ref,
                     h_ref_o, t1_ref, t2_ref):
    agg = agg0_ref[0] + agg1_ref[0]
    hn = _silu(jnp.dot(h_ref[...], w1_ref[...],
                       preferred_element_type=jnp.float32)
               + jnp.dot(agg, w2_ref[...],
                         preferred_element_type=jnp.float32)
               + nb_ref[...])
    h_ref_o[...] = hn
    bc = bc_ref[...]
    xi = 0.25 * jax.lax.broadcasted_iota(
        jnp.int32, (1, NPTS), 1).astype(jnp.float32)
    mi = 1.0 - bc * (1.0 - xi)
    mj = 1.0 - bc * xi
    ha = jnp.dot(hn, wi_ref[...], preferred_element_type=jnp.float32)
    hb = jnp.dot(hn, wj_ref[...], preferred_element_type=jnp.float32)
    zpad = jnp.zeros((hn.shape[0], 128 - 64 - NPTS), jnp.float32)
    t1_ref[...] = jnp.concatenate([ha, mi, zpad], axis=1)
    t2_ref[...] = jnp.concatenate([hb, mj, zpad], axis=1)


def _node_final(h, aggp, nw1, nw2, nb, wi, wj, bc):
    full = lambda i: (0, 0)
    return pl.pallas_call(
        _node_final_body,
        grid=(NP // BN,),
        in_specs=[
            pl.BlockSpec((BN, H), lambda i: (i, 0)),
            pl.BlockSpec((1, BN, H), lambda i: (0, i, 0)),
            pl.BlockSpec((1, BN, H), lambda i: (1, i, 0)),
            pl.BlockSpec((H, H), full),
            pl.BlockSpec((H, H), full),
            pl.BlockSpec((1, H), full),
            pl.BlockSpec((H, 64), full),
            pl.BlockSpec((H, 64), full),
            pl.BlockSpec((BN, 1), lambda i: (i, 0)),
        ],
        out_specs=[
            pl.BlockSpec((BN, H), lambda i: (i, 0)),
            pl.BlockSpec((BN, 128), lambda i: (i, 0)),
            pl.BlockSpec((BN, 128), lambda i: (i, 0)),
        ],
        out_shape=[
            jax.ShapeDtypeStruct((NP, H), jnp.float32),
            jax.ShapeDtypeStruct((NP, 128), jnp.float32),
            jax.ShapeDtypeStruct((NP, 128), jnp.float32),
        ],
    )(h, aggp, aggp, nw1, nw2, nb, wi, wj, bc)


def _field_body(g_ref, w0_ref, b1_ref, w2_ref, b2_ref, w3_ref,
                b3_ref, f_ref):
    g = g_ref[...]
    base = g[:, :64]
    w0 = w0_ref[...]
    for k in range(NPTS):
        z = _silu(base + _XI[k] * w0 + b1_ref[...])
        z = _silu(jnp.dot(z, w2_ref[...],
                          preferred_element_type=jnp.float32) + b2_ref[...])
        f = jnp.dot(z, w3_ref[...], preferred_element_type=jnp.float32) \
            + b3_ref[...]
        mask = g[:, 64 + k][:, None]
        f_ref[:, 3 * k:3 * k + 3] = f * mask


def _field(g, w0, b1, w2, b2, w3, b3):
    full = lambda i: (0, 0)
    return pl.pallas_call(
        _field_body,
        grid=(EP // BE,),
        in_specs=[
            pl.BlockSpec((BE, 128), lambda i: (i, 0)),
            pl.BlockSpec((1, 64), full),
            pl.BlockSpec((1, 64), full),
            pl.BlockSpec((64, 64), full),
            pl.BlockSpec((1, 64), full),
            pl.BlockSpec((64, 3), full),
            pl.BlockSpec((1, 3), full),
        ],
        out_specs=pl.BlockSpec((BE, NPTS * 3), lambda i: (i, 0)),
        out_shape=jax.ShapeDtypeStruct((EP, NPTS * 3), jnp.float32),
    )(g, w0, b1, w2, b2, w3, b3)


def kernel(x, edge_index, edge_attr, connectivity, bc_disp, prop_I22,
           enc_W1, enc_b1, enc_W2, enc_b2,
           conv0_eW, conv0_eb, conv0_nW, conv0_nb,
           conv1_eW, conv1_eb, conv1_nW, conv1_nb,
           conv2_eW, conv2_eb, conv2_nW, conv2_nb,
           ef_W1, ef_b1, ef_W2, ef_b2, ef_W3, ef_b3):
    convs = [(conv0_eW, conv0_eb, conv0_nW, conv0_nb),
             (conv1_eW, conv1_eb, conv1_nW, conv1_nb),
             (conv2_eW, conv2_eb, conv2_nW, conv2_nb)]

    x_p = jnp.pad(x, ((0, NP - N), (0, 0)))
    bc_p = jnp.pad(bc_disp, ((0, NP - N), (0, 0)))
    src = jnp.pad(edge_index[0], (0, EP - E))
    dst = jnp.pad(edge_index[1], (0, EP - E), constant_values=N)
    ea_p = jnp.pad(edge_attr, ((0, EP - E), (0, 0)))
    n1 = jnp.pad(connectivity[:, 0], (0, EP - E))
    n2 = jnp.pad(connectivity[:, 1], (0, EP - E))
    i2 = jnp.stack([src.reshape(TILES, NCCHUNK, CCH),
                    dst.reshape(TILES, NCCHUNK, CCH)], axis=2)
    i2f = jnp.stack([n1.reshape(TILES, NF, CHF),
                     n2.reshape(TILES, NF, CHF)], axis=2)

    r2 = lambda v: v.reshape(1, -1)

    h, a, b = _encode(x_p, enc_W1, r2(enc_b1), enc_W2, r2(enc_b2),
                      convs[0][0][:H], convs[0][0][H:2 * H])

    for li in range(3):
        eW, eb, nW, nb = convs[li]
        c = _edge_c(ea_p, eW[2 * H:], r2(eb))
        aggp = _conv_sc(a, b, c, i2)
        if li < 2:
            nws, nwd = convs[li + 1][0][:H], convs[li + 1][0][H:2 * H]
            h, a, b = _node_update(h, aggp, nW[:H], nW[H:], r2(nb), nws, nwd)
        else:
            h, t1, t2 = _node_final(h, aggp, nW[:H], nW[H:], r2(nb),
                                    ef_W1[1:1 + H], ef_W1[1 + H:], bc_p)

    g = _field_sc(t1, t2, i2f)
    f = _field(g, r2(ef_W1[0]), r2(ef_b1), ef_W2, r2(ef_b2),
               ef_W3, r2(ef_b3))

    fields = f[:E].reshape(E, NPTS, 3)
    xi = jnp.broadcast_to(
        jnp.asarray(_XI, jnp.float32)[None, :, None], (E, NPTS, 1))
    return h[:N], xi, fields, prop_I22


# speedup vs baseline: 5.4256x; 1.2588x over previous
import functools

import jax
import jax.numpy as jnp
from jax import lax
from jax.experimental import pallas as pl
from jax.experimental.pallas import tpu as pltpu
from jax.experimental.pallas import tpu_sc as plsc

N = 10000
E = 160000
H = 128
NODE_IN = 9
EDGE_DIM = 11
NPTS = 5

BN = 2000
BE = 2000

_XI = [0.0, 0.25, 0.5, 0.75, 1.0]

NC = 2
NS = 16
NPA = 10112
RPSA = NPA // NS

CCH = 40
PT0, PT1 = 6000, 4000
NCH0, NCH1 = PT0 // CCH, PT1 // CCH
C1OFF = NS * PT0

CHF = 40
PF0, PF1 = 7600, 2400
NF0, NF1 = PF0 // CHF, PF1 // CHF
F1OFF = NS * PF0

_SC_MESH = plsc.VectorSubcoreMesh(core_axis_name="c", subcore_axis_name="s")


def _silu(v):
    return v * jax.nn.sigmoid(v)


def _conv_sc(a, b, c, i2a, i2b):
    @functools.partial(
        pl.kernel,
        out_type=jax.ShapeDtypeStruct((NC, NPA, H), jnp.float32),
        mesh=_SC_MESH,
        scratch_types=[
            pltpu.VMEM((2, CCH), jnp.int32),
            pltpu.VMEM((2, CCH), jnp.int32),
            pltpu.VMEM((CCH, H), jnp.float32),
            pltpu.VMEM((CCH, H), jnp.float32),
            pltpu.VMEM((CCH, H), jnp.float32),
            pltpu.VMEM((CCH, H), jnp.float32),
            pltpu.VMEM((CCH, H), jnp.float32),
            pltpu.VMEM((CCH, H), jnp.float32),
            pltpu.VMEM_SHARED((NPA, H), jnp.float32),
            pltpu.SemaphoreType.DMA,
            pltpu.SemaphoreType.DMA,
            pltpu.SemaphoreType.DMA,
            pltpu.SemaphoreType.DMA,
            pltpu.SemaphoreType.DMA,
            pltpu.SemaphoreType.DMA,
        ],
    )
    def k(a_hbm, b_hbm, c_hbm, i2a_hbm, i2b_hbm, out_hbm,
          ib0, ib1, ar0, ar1, br0, br1, cr0, cr1, acc,
          sa0, sa1, sb0, sb1, sc0, sc1):
        cid = lax.axis_index("c")
        sid = lax.axis_index("s")
        zrow = sid * RPSA

        @pl.loop(0, CCH)
        def _(r):
            @pl.loop(0, H, step=16)
            def _(cc):
                ar0[r, pl.ds(cc, 16)] = jnp.zeros((16,), jnp.float32)

        @pl.loop(0, RPSA - RPSA % CCH, step=CCH)
        def _(rr):
            pltpu.sync_copy(ar0, acc.at[pl.ds(zrow + rr, CCH)])
        pltpu.sync_copy(ar0.at[pl.ds(0, RPSA % CCH)],
                        acc.at[pl.ds(zrow + RPSA - RPSA % CCH, RPSA % CCH)])

        plsc.subcore_barrier()

        bufs = ((ib0, ar0, br0, cr0, sa0, sb0, sc0),
                (ib1, ar1, br1, cr1, sa1, sb1, sc1))

        def run(base, nch, i2_hbm):
            def issue(ci, ib, ar, br, cr, sa, sb, sc):
                off = base + ci * CCH
                pltpu.sync_copy(i2_hbm.at[sid, ci], ib)
                pltpu.async_copy(a_hbm.at[ib.at[0]], ar, sa)
                pltpu.async_copy(b_hbm.at[ib.at[1]], br, sb)
                pltpu.async_copy(c_hbm.at[pl.ds(off, CCH)], cr, sc)

            issue(0, *bufs[0])
            issue(1, *bufs[1])

            @pl.loop(0, nch, step=2)
            def _(g):
                for p in range(2):
                    ib, ar, br, cr, sa, sb, sc = bufs[p]
                    ci = g + p
                    pltpu.make_async_copy(a_hbm.at[ib.at[0]], ar, sa).wait()
                    pltpu.make_async_copy(b_hbm.at[ib.at[1]], br, sb).wait()
                    pltpu.make_async_copy(c_hbm.at[pl.ds(base, CCH)], cr,
                                          sc).wait()

                    @pl.loop(0, CCH)
                    def _(r):
                        @pl.loop(0, H, step=16)
                        def _(cc):
                            v = (ar[r, pl.ds(cc, 16)] + br[r, pl.ds(cc, 16)]
                                 + cr[r, pl.ds(cc, 16)])
                            ar[r, pl.ds(cc, 16)] = v / (1.0 + jnp.exp(-v))

                    pltpu.sync_copy(ar, acc.at[ib.at[1]], add=True)

                    @pl.when(ci + 2 < nch)
                    def _():
                        issue(ci + 2, *bufs[p])

        @pl.when(cid == 0)
        def _():
            run(sid * PT0, NCH0, i2a_hbm)

        @pl.when(cid == 1)
        def _():
            run(C1OFF + sid * PT1, NCH1, i2b_hbm)

        plsc.subcore_barrier()
        pltpu.sync_copy(acc.at[pl.ds(zrow, RPSA)],
                        out_hbm.at[cid, pl.ds(zrow, RPSA)])

    return k(a, b, c, i2a, i2b)


def _field_sc(t1, t2, f2a, f2b):
    @functools.partial(
        pl.kernel,
        out_type=jax.ShapeDtypeStruct((E, 128), jnp.float32),
        mesh=_SC_MESH,
        scratch_types=[
            pltpu.VMEM((2, CHF), jnp.int32),
            pltpu.VMEM((2, CHF), jnp.int32),
            pltpu.VMEM((CHF, 128), jnp.float32),
            pltpu.VMEM((CHF, 128), jnp.float32),
            pltpu.VMEM((CHF, 128), jnp.float32),
            pltpu.VMEM((CHF, 128), jnp.float32),
            pltpu.VMEM((CHF, 128), jnp.float32),
            pltpu.VMEM((CHF, 128), jnp.float32),
            pltpu.SemaphoreType.DMA,
            pltpu.SemaphoreType.DMA,
            pltpu.SemaphoreType.DMA,
            pltpu.SemaphoreType.DMA,
            pltpu.SemaphoreType.DMA,
            pltpu.SemaphoreType.DMA,
        ],
    )
    def k(t1_hbm, t2_hbm, f2a_hbm, f2b_hbm, out_hbm,
          ib0, ib1, g10, g11, g20, g21, ob0, ob1,
          s10, s11, s20, s21, so0, so1):
        cid = lax.axis_index("c")
        sid = lax.axis_index("s")

        bufs = ((ib0, g10, g20, ob0, s10, s20, so0),
                (ib1, g11, g21, ob1, s11, s21, so1))

        def run(base, nf, i2_hbm):
            def issue(ci, ib, g1, g2, ob, s1, s2, so):
                off = base + ci * CHF
                pltpu.sync_copy(i2_hbm.at[sid, ci], ib)
                pltpu.async_copy(t1_hbm.at[ib.at[0]], g1, s1)
                pltpu.async_copy(t2_hbm.at[ib.at[1]], g2, s2)

            issue(0, *bufs[0])
            issue(1, *bufs[1])

            @pl.loop(0, nf, step=2)
            def _(g):
                for p in range(2):
                    ib, g1, g2, ob, s1, s2, so = bufs[p]
                    ci = g + p
                    pltpu.make_async_copy(t1_hbm.at[ib.at[0]], g1, s1).wait()
                    pltpu.make_async_copy(t2_hbm.at[ib.at[1]], g2, s2).wait()

                    @pl.when(ci >= 2)
                    def _():
                        pltpu.make_async_copy(
                            ob, out_hbm.at[pl.ds(base, CHF)], so).wait()

                    @pl.loop(0, CHF)
                    def _(r):
                        @pl.loop(0, 64, step=16)
                        def _(cc):
                            ob[r, pl.ds(cc, 16)] = (g1[r, pl.ds(cc, 16)]
                                                    + g2[r, pl.ds(cc, 16)])
                        ob[r, pl.ds(64, 16)] = (g1[r, pl.ds(64, 16)]
                                                * g2[r, pl.ds(64, 16)])

                    pltpu.async_copy(
                        ob, out_hbm.at[pl.ds(base + ci * CHF, CHF)], so)

                    @pl.when(ci + 2 < nf)
                    def _():
                        issue(ci + 2, *bufs[p])

            pltpu.make_async_copy(ob0, out_hbm.at[pl.ds(base, CHF)],
                                  so0).wait()
            pltpu.make_async_copy(ob1, out_hbm.at[pl.ds(base, CHF)],
                                  so1).wait()

        @pl.when(cid == 0)
        def _():
            run(sid * PF0, NF0, f2a_hbm)

        @pl.when(cid == 1)
        def _():
            run(F1OFF + sid * PF1, NF1, f2b_hbm)

    return k(t1, t2, f2a, f2b)


def _enc_body(x_ref, w1_ref, b1_ref, w2_ref, b2_ref, ws_ref, wd_ref,
              h_ref, a_ref, b_ref):
    h1 = _silu(jnp.dot(x_ref[...], w1_ref[...],
                       preferred_element_type=jnp.float32) + b1_ref[...])
    h2 = _silu(jnp.dot(h1, w2_ref[...],
                       preferred_element_type=jnp.float32) + b2_ref[...])
    h_ref[...] = h2
    a_ref[...] = jnp.dot(h2, ws_ref[...], preferred_element_type=jnp.float32)
    b_ref[...] = jnp.dot(h2, wd_ref[...], preferred_element_type=jnp.float32)


def _encode(x, w1, b1, w2, b2, ws, wd):
    full = lambda i: (0, 0)
    return pl.pallas_call(
        _enc_body,
        grid=(N // BN,),
        in_specs=[
            pl.BlockSpec((BN, NODE_IN), lambda i: (i, 0)),
            pl.BlockSpec((NODE_IN, H), full),
            pl.BlockSpec((1, H), full),
            pl.BlockSpec((H, H), full),
            pl.BlockSpec((1, H), full),
            pl.BlockSpec((H, H), full),
            pl.BlockSpec((H, H), full),
        ],
        out_specs=[pl.BlockSpec((BN, H), lambda i: (i, 0))] * 3,
        out_shape=[jax.ShapeDtypeStruct((N, H), jnp.float32)] * 3,
    )(x, w1, b1, w2, b2, ws, wd)


def _edge_c_body(ea_ref, wa_ref, eb_ref, c_ref):
    c_ref[...] = jnp.dot(ea_ref[...], wa_ref[...],
                         preferred_element_type=jnp.float32) + eb_ref[...]


def _edge_c(ea, wa, eb):
    return pl.pallas_call(
        _edge_c_body,
        grid=(E // BE,),
        in_specs=[
            pl.BlockSpec((BE, EDGE_DIM), lambda i: (i, 0)),
            pl.BlockSpec((EDGE_DIM, H), lambda i: (0, 0)),
            pl.BlockSpec((1, H), lambda i: (0, 0)),
        ],
        out_specs=pl.BlockSpec((BE, H), lambda i: (i, 0)),
        out_shape=jax.ShapeDtypeStruct((E, H), jnp.float32),
    )(ea, wa, eb)


def _node_upd_body(h_ref, agg0_ref, agg1_ref, w1_ref, w2_ref, nb_ref,
                   ws_ref, wd_ref, h_ref_o, a_ref, b_ref):
    agg = agg0_ref[0] + agg1_ref[0]
    hn = _silu(jnp.dot(h_ref[...], w1_ref[...],
                       preferred_element_type=jnp.float32)
               + jnp.dot(agg, w2_ref[...],
                         preferred_element_type=jnp.float32)
               + nb_ref[...])
    h_ref_o[...] = hn
    a_ref[...] = jnp.dot(hn, ws_ref[...], preferred_element_type=jnp.float32)
    b_ref[...] = jnp.dot(hn, wd_ref[...], preferred_element_type=jnp.float32)


def _node_update(h, aggp, nw1, nw2, nb, ws, wd):
    full = lambda i: (0, 0)
    return pl.pallas_call(
        _node_upd_body,
        grid=(N // BN,),
        in_specs=[
            pl.BlockSpec((BN, H), lambda i: (i, 0)),
            pl.BlockSpec((1, BN, H), lambda i: (0, i, 0)),
            pl.BlockSpec((1, BN, H), lambda i: (1, i, 0)),
            pl.BlockSpec((H, H), full),
            pl.BlockSpec((H, H), full),
            pl.BlockSpec((1, H), full),
            pl.BlockSpec((H, H), full),
            pl.BlockSpec((H, H), full),
        ],
        out_specs=[pl.BlockSpec((BN, H), lambda i: (i, 0))] * 3,
        out_shape=[jax.ShapeDtypeStruct((N, H), jnp.float32)] * 3,
    )(h, aggp, aggp, nw1, nw2, nb, ws, wd)


def _node_final_body(h_ref, agg0_ref, agg1_ref, w1_ref, w2_ref, nb_ref,
                     wi_ref, wj_ref, bc_ref,
                     h_ref_o, t1_ref, t2_ref):
    agg = agg0_ref[0] + agg1_ref[0]
    hn = _silu(jnp.dot(h_ref[...], w1_ref[...],
                       preferred_element_type=jnp.float32)
               + jnp.dot(agg, w2_ref[...],
                         preferred_element_type=jnp.float32)
               + nb_ref[...])
    h_ref_o[...] = hn
    bc = bc_ref[...]
    xi = 0.25 * lax.broadcasted_iota(jnp.int32, (1, NPTS), 1).astype(
        jnp.float32)
    mi = 1.0 - bc * (1.0 - xi)
    mj = 1.0 - bc * xi
    ha = jnp.dot(hn, wi_ref[...], preferred_element_type=jnp.float32)
    hb = jnp.dot(hn, wj_ref[...], preferred_element_type=jnp.float32)
    zpad = jnp.zeros((hn.shape[0], 128 - 64 - NPTS), jnp.float32)
    t1_ref[...] = jnp.concatenate([ha, mi, zpad], axis=1)
    t2_ref[...] = jnp.concatenate([hb, mj, zpad], axis=1)


def _node_final(h, aggp, nw1, nw2, nb, wi, wj, bc):
    full = lambda i: (0, 0)
    return pl.pallas_call(
        _node_final_body,
        grid=(N // BN,),
        in_specs=[
            pl.BlockSpec((BN, H), lambda i: (i, 0)),
            pl.BlockSpec((1, BN, H), lambda i: (0, i, 0)),
            pl.BlockSpec((1, BN, H), lambda i: (1, i, 0)),
            pl.BlockSpec((H, H), full),
            pl.BlockSpec((H, H), full),
            pl.BlockSpec((1, H), full),
            pl.BlockSpec((H, 64), full),
            pl.BlockSpec((H, 64), full),
            pl.BlockSpec((BN, 1), lambda i: (i, 0)),
        ],
        out_specs=[
            pl.BlockSpec((BN, H), lambda i: (i, 0)),
            pl.BlockSpec((BN, 128), lambda i: (i, 0)),
            pl.BlockSpec((BN, 128), lambda i: (i, 0)),
        ],
        out_shape=[
            jax.ShapeDtypeStruct((N, H), jnp.float32),
            jax.ShapeDtypeStruct((N, 128), jnp.float32),
            jax.ShapeDtypeStruct((N, 128), jnp.float32),
        ],
    )(h, aggp, aggp, nw1, nw2, nb, wi, wj, bc)


def _field_body(g_ref, w0_ref, b1_ref, w2_ref, b2_ref, w3s_ref, r_ref,
                b3t_ref, f_ref):
    g = g_ref[...]
    base = g[:, :64]
    w0 = w0_ref[...]
    acc = None
    for k in range(NPTS):
        z = _silu(base + _XI[k] * w0 + b1_ref[...])
        z = _silu(jnp.dot(z, w2_ref[...],
                          preferred_element_type=jnp.float32) + b2_ref[...])
        t = jnp.dot(z, w3s_ref[64 * k:64 * (k + 1), :],
                    preferred_element_type=jnp.float32)
        acc = t if acc is None else acc + t
    mask15 = jnp.dot(g[:, 64:64 + NPTS], r_ref[...],
                     preferred_element_type=jnp.float32)
    f_ref[...] = (acc + b3t_ref[...]) * mask15


def _field(g, w0, b1, w2, b2, w3s, r, b3t):
    full = lambda i: (0, 0)
    return pl.pallas_call(
        _field_body,
        grid=(E // BE,),
        in_specs=[
            pl.BlockSpec((BE, 128), lambda i: (i, 0)),
            pl.BlockSpec((1, 64), full),
            pl.BlockSpec((1, 64), full),
            pl.BlockSpec((64, 64), full),
            pl.BlockSpec((1, 64), full),
            pl.BlockSpec((NPTS * 64, NPTS * 3), full),
            pl.BlockSpec((NPTS, NPTS * 3), full),
            pl.BlockSpec((1, NPTS * 3), full),
        ],
        out_specs=pl.BlockSpec((BE, NPTS * 3), lambda i: (i, 0)),
        out_shape=jax.ShapeDtypeStruct((E, NPTS * 3), jnp.float32),
    )(g, w0, b1, w2, b2, w3s, r, b3t)


def kernel(x, edge_index, edge_attr, connectivity, bc_disp, prop_I22,
           enc_W1, enc_b1, enc_W2, enc_b2,
           conv0_eW, conv0_eb, conv0_nW, conv0_nb,
           conv1_eW, conv1_eb, conv1_nW, conv1_nb,
           conv2_eW, conv2_eb, conv2_nW, conv2_nb,
           ef_W1, ef_b1, ef_W2, ef_b2, ef_W3, ef_b3):
    convs = [(conv0_eW, conv0_eb, conv0_nW, conv0_nb),
             (conv1_eW, conv1_eb, conv1_nW, conv1_nb),
             (conv2_eW, conv2_eb, conv2_nW, conv2_nb)]

    r2 = lambda v: v.reshape(1, -1)

    def _split(svec, dvec, cut, pt0, pt1, ch):
        ia = jnp.stack([svec[:cut].reshape(NS, pt0 // ch, ch),
                        dvec[:cut].reshape(NS, pt0 // ch, ch)], axis=2)
        ib_ = jnp.stack([svec[cut:].reshape(NS, pt1 // ch, ch),
                         dvec[cut:].reshape(NS, pt1 // ch, ch)], axis=2)
        return ia, ib_

    i2a, i2b = _split(edge_index[0], edge_index[1], C1OFF, PT0, PT1, CCH)
    f2a, f2b = _split(connectivity[:, 0], connectivity[:, 1],
                      F1OFF, PF0, PF1, CHF)

    w3s = jnp.zeros((NPTS * 64, NPTS * 3), jnp.float32)
    rrep = jnp.zeros((NPTS, NPTS * 3), jnp.float32)
    for k in range(NPTS):
        w3s = w3s.at[64 * k:64 * (k + 1), 3 * k:3 * k + 3].set(ef_W3)
        rrep = rrep.at[k, 3 * k:3 * k + 3].set(1.0)
    b3t = jnp.tile(ef_b3, NPTS)[None, :]

    h, a, b = _encode(x, enc_W1, r2(enc_b1), enc_W2, r2(enc_b2),
                      convs[0][0][:H], convs[0][0][H:2 * H])

    for li in range(3):
        eW, eb, nW, nb = convs[li]
        c = _edge_c(edge_attr, eW[2 * H:], r2(eb))
        aggp = _conv_sc(a, b, c, i2a, i2b)
        if li < 2:
            nws, nwd = convs[li + 1][0][:H], convs[li + 1][0][H:2 * H]
            h, a, b = _node_update(h, aggp, nW[:H], nW[H:], r2(nb), nws, nwd)
        else:
            h, t1, t2 = _node_final(h, aggp, nW[:H], nW[H:], r2(nb),
                                    ef_W1[1:1 + H], ef_W1[1 + H:], bc_disp)

    g = _field_sc(t1, t2, f2a, f2b)
    f = _field(g, r2(ef_W1[0]), r2(ef_b1), ef_W2, r2(ef_b2), w3s, rrep, b3t)

    fields = f.reshape(E, NPTS, 3)
    xi = jnp.broadcast_to(
        jnp.asarray(_XI, jnp.float32)[None, :, None], (E, NPTS, 1))
    return h, xi, fields, prop_I22


# speedup vs baseline: 5.6039x; 1.0329x over previous
import functools

import jax
import jax.numpy as jnp
from jax import lax
from jax.experimental import pallas as pl
from jax.experimental.pallas import tpu as pltpu
from jax.experimental.pallas import tpu_sc as plsc

N = 10000
E = 160000
H = 128
NODE_IN = 9
EDGE_DIM = 11
NPTS = 5

BN = 2000
BE = 2000

_XI = [0.0, 0.25, 0.5, 0.75, 1.0]

NC = 2
NS = 16
NPA = 10112
RPSA = NPA // NS

CCH = 40
PT0, PT1 = 5440, 4560
NCH0, NCH1 = PT0 // CCH, PT1 // CCH
C1OFF = NS * PT0

CHF = 40
PF0, PF1 = 6400, 3600
NF0, NF1 = PF0 // CHF, PF1 // CHF
F1OFF = NS * PF0

_SC_MESH = plsc.VectorSubcoreMesh(core_axis_name="c", subcore_axis_name="s")


def _silu(v):
    return v * jax.nn.sigmoid(v)


def _conv_sc(a, b, c, i2a, i2b):
    @functools.partial(
        pl.kernel,
        out_type=jax.ShapeDtypeStruct((NC, NPA, H), jnp.float32),
        mesh=_SC_MESH,
        scratch_types=[
            pltpu.VMEM((2, CCH), jnp.int32),
            pltpu.VMEM((2, CCH), jnp.int32),
            pltpu.VMEM((CCH, H), jnp.float32),
            pltpu.VMEM((CCH, H), jnp.float32),
            pltpu.VMEM((CCH, H), jnp.float32),
            pltpu.VMEM((CCH, H), jnp.float32),
            pltpu.VMEM((CCH, H), jnp.float32),
            pltpu.VMEM((CCH, H), jnp.float32),
            pltpu.VMEM_SHARED((NPA, H), jnp.float32),
            pltpu.SemaphoreType.DMA,
            pltpu.SemaphoreType.DMA,
            pltpu.SemaphoreType.DMA,
            pltpu.SemaphoreType.DMA,
            pltpu.SemaphoreType.DMA,
            pltpu.SemaphoreType.DMA,
        ],
    )
    def k(a_hbm, b_hbm, c_hbm, i2a_hbm, i2b_hbm, out_hbm,
          ib0, ib1, ar0, ar1, br0, br1, cr0, cr1, acc,
          sa0, sa1, sb0, sb1, sc0, sc1):
        cid = lax.axis_index("c")
        sid = lax.axis_index("s")
        zrow = sid * RPSA

        @pl.loop(0, CCH)
        def _(r):
            @pl.loop(0, H, step=16)
            def _(cc):
                ar0[r, pl.ds(cc, 16)] = jnp.zeros((16,), jnp.float32)

        @pl.loop(0, RPSA - RPSA % CCH, step=CCH)
        def _(rr):
            pltpu.sync_copy(ar0, acc.at[pl.ds(zrow + rr, CCH)])
        pltpu.sync_copy(ar0.at[pl.ds(0, RPSA % CCH)],
                        acc.at[pl.ds(zrow + RPSA - RPSA % CCH, RPSA % CCH)])

        plsc.subcore_barrier()

        bufs = ((ib0, ar0, br0, cr0, sa0, sb0, sc0),
                (ib1, ar1, br1, cr1, sa1, sb1, sc1))

        def run(base, nch, i2_hbm):
            def issue(ci, ib, ar, br, cr, sa, sb, sc):
                off = base + ci * CCH
                pltpu.sync_copy(i2_hbm.at[sid, ci], ib)
                pltpu.async_copy(a_hbm.at[ib.at[0]], ar, sa)
                pltpu.async_copy(b_hbm.at[ib.at[1]], br, sb)
                pltpu.async_copy(c_hbm.at[pl.ds(off, CCH)], cr, sc)

            issue(0, *bufs[0])
            issue(1, *bufs[1])

            @pl.loop(0, nch, step=2)
            def _(g):
                for p in range(2):
                    ib, ar, br, cr, sa, sb, sc = bufs[p]
                    ci = g + p
                    pltpu.make_async_copy(a_hbm.at[ib.at[0]], ar, sa).wait()
                    pltpu.make_async_copy(b_hbm.at[ib.at[1]], br, sb).wait()
                    pltpu.make_async_copy(c_hbm.at[pl.ds(base, CCH)], cr,
                                          sc).wait()

                    @pl.loop(0, CCH)
                    def _(r):
                        @pl.loop(0, H, step=16)
                        def _(cc):
                            v = (ar[r, pl.ds(cc, 16)] + br[r, pl.ds(cc, 16)]
                                 + cr[r, pl.ds(cc, 16)])
                            ar[r, pl.ds(cc, 16)] = v / (1.0 + jnp.exp(-v))

                    pltpu.sync_copy(ar, acc.at[ib.at[1]], add=True)

                    @pl.when(ci + 2 < nch)
                    def _():
                        issue(ci + 2, *bufs[p])

        @pl.when(cid == 0)
        def _():
            run(sid * PT0, NCH0, i2a_hbm)

        @pl.when(cid == 1)
        def _():
            run(C1OFF + sid * PT1, NCH1, i2b_hbm)

        plsc.subcore_barrier()
        pltpu.sync_copy(acc.at[pl.ds(zrow, RPSA)],
                        out_hbm.at[cid, pl.ds(zrow, RPSA)])

    return k(a, b, c, i2a, i2b)


def _field_sc(t1, t2, f2a, f2b):
    @functools.partial(
        pl.kernel,
        out_type=jax.ShapeDtypeStruct((E, 128), jnp.float32),
        mesh=_SC_MESH,
        scratch_types=[
            pltpu.VMEM((2, CHF), jnp.int32),
            pltpu.VMEM((2, CHF), jnp.int32),
            pltpu.VMEM((CHF, 128), jnp.float32),
            pltpu.VMEM((CHF, 128), jnp.float32),
            pltpu.VMEM((CHF, 128), jnp.float32),
            pltpu.VMEM((CHF, 128), jnp.float32),
            pltpu.VMEM((CHF, 128), jnp.float32),
            pltpu.VMEM((CHF, 128), jnp.float32),
            pltpu.SemaphoreType.DMA,
            pltpu.SemaphoreType.DMA,
            pltpu.SemaphoreType.DMA,
            pltpu.SemaphoreType.DMA,
            pltpu.SemaphoreType.DMA,
            pltpu.SemaphoreType.DMA,
        ],
    )
    def k(t1_hbm, t2_hbm, f2a_hbm, f2b_hbm, out_hbm,
          ib0, ib1, g10, g11, g20, g21, ob0, ob1,
          s10, s11, s20, s21, so0, so1):
        cid = lax.axis_index("c")
        sid = lax.axis_index("s")

        bufs = ((ib0, g10, g20, ob0, s10, s20, so0),
                (ib1, g11, g21, ob1, s11, s21, so1))

        def run(base, nf, i2_hbm):
            def issue(ci, ib, g1, g2, ob, s1, s2, so):
                off = base + ci * CHF
                pltpu.sync_copy(i2_hbm.at[sid, ci], ib)
                pltpu.async_copy(t1_hbm.at[ib.at[0]], g1, s1)
                pltpu.async_copy(t2_hbm.at[ib.at[1]], g2, s2)

            issue(0, *bufs[0])
            issue(1, *bufs[1])

            @pl.loop(0, nf, step=2)
            def _(g):
                for p in range(2):
                    ib, g1, g2, ob, s1, s2, so = bufs[p]
                    ci = g + p
                    pltpu.make_async_copy(t1_hbm.at[ib.at[0]], g1, s1).wait()
                    pltpu.make_async_copy(t2_hbm.at[ib.at[1]], g2, s2).wait()

                    @pl.when(ci >= 2)
                    def _():
                        pltpu.make_async_copy(
                            ob, out_hbm.at[pl.ds(base, CHF)], so).wait()

                    @pl.loop(0, CHF)
                    def _(r):
                        @pl.loop(0, 64, step=16)
                        def _(cc):
                            ob[r, pl.ds(cc, 16)] = (g1[r, pl.ds(cc, 16)]
                                                    + g2[r, pl.ds(cc, 16)])
                        ob[r, pl.ds(64, 16)] = (g1[r, pl.ds(64, 16)]
                                                * g2[r, pl.ds(64, 16)])

                    pltpu.async_copy(
                        ob, out_hbm.at[pl.ds(base + ci * CHF, CHF)], so)

                    @pl.when(ci + 2 < nf)
                    def _():
                        issue(ci + 2, *bufs[p])

            pltpu.make_async_copy(ob0, out_hbm.at[pl.ds(base, CHF)],
                                  so0).wait()
            pltpu.make_async_copy(ob1, out_hbm.at[pl.ds(base, CHF)],
                                  so1).wait()

        @pl.when(cid == 0)
        def _():
            run(sid * PF0, NF0, f2a_hbm)

        @pl.when(cid == 1)
        def _():
            run(F1OFF + sid * PF1, NF1, f2b_hbm)

    return k(t1, t2, f2a, f2b)


def _enc_body(x_ref, w1_ref, b1_ref, w2_ref, b2_ref, ws_ref, wd_ref,
              h_ref, a_ref, b_ref):
    h1 = _silu(jnp.dot(x_ref[...], w1_ref[...],
                       preferred_element_type=jnp.float32) + b1_ref[...])
    h2 = _silu(jnp.dot(h1, w2_ref[...],
                       preferred_element_type=jnp.float32) + b2_ref[...])
    h_ref[...] = h2
    a_ref[...] = jnp.dot(h2, ws_ref[...], preferred_element_type=jnp.float32)
    b_ref[...] = jnp.dot(h2, wd_ref[...], preferred_element_type=jnp.float32)


def _encode(x, w1, b1, w2, b2, ws, wd):
    full = lambda i: (0, 0)
    return pl.pallas_call(
        _enc_body,
        grid=(N // BN,),
        in_specs=[
            pl.BlockSpec((BN, NODE_IN), lambda i: (i, 0)),
            pl.BlockSpec((NODE_IN, H), full),
            pl.BlockSpec((1, H), full),
            pl.BlockSpec((H, H), full),
            pl.BlockSpec((1, H), full),
            pl.BlockSpec((H, H), full),
            pl.BlockSpec((H, H), full),
        ],
        out_specs=[pl.BlockSpec((BN, H), lambda i: (i, 0))] * 3,
        out_shape=[jax.ShapeDtypeStruct((N, H), jnp.float32)] * 3,
    )(x, w1, b1, w2, b2, ws, wd)


def _edge_c_body(ea_ref, wa_ref, eb_ref, c_ref):
    c_ref[...] = jnp.dot(ea_ref[...], wa_ref[...],
                         preferred_element_type=jnp.float32) + eb_ref[...]


def _edge_c(ea, wa, eb):
    return pl.pallas_call(
        _edge_c_body,
        grid=(E // BE,),
        in_specs=[
            pl.BlockSpec((BE, EDGE_DIM), lambda i: (i, 0)),
            pl.BlockSpec((EDGE_DIM, H), lambda i: (0, 0)),
            pl.BlockSpec((1, H), lambda i: (0, 0)),
        ],
        out_specs=pl.BlockSpec((BE, H), lambda i: (i, 0)),
        out_shape=jax.ShapeDtypeStruct((E, H), jnp.float32),
    )(ea, wa, eb)


def _node_upd_body(h_ref, agg0_ref, agg1_ref, w1_ref, w2_ref, nb_ref,
                   ws_ref, wd_ref, h_ref_o, a_ref, b_ref):
    agg = agg0_ref[0] + agg1_ref[0]
    hn = _silu(jnp.dot(h_ref[...], w1_ref[...],
                       preferred_element_type=jnp.float32)
               + jnp.dot(agg, w2_ref[...],
                         preferred_element_type=jnp.float32)
               + nb_ref[...])
    h_ref_o[...] = hn
    a_ref[...] = jnp.dot(hn, ws_ref[...], preferred_element_type=jnp.float32)
    b_ref[...] = jnp.dot(hn, wd_ref[...], preferred_element_type=jnp.float32)


def _node_update(h, aggp, nw1, nw2, nb, ws, wd):
    full = lambda i: (0, 0)
    return pl.pallas_call(
        _node_upd_body,
        grid=(N // BN,),
        in_specs=[
            pl.BlockSpec((BN, H), lambda i: (i, 0)),
            pl.BlockSpec((1, BN, H), lambda i: (0, i, 0)),
            pl.BlockSpec((1, BN, H), lambda i: (1, i, 0)),
            pl.BlockSpec((H, H), full),
            pl.BlockSpec((H, H), full),
            pl.BlockSpec((1, H), full),
            pl.BlockSpec((H, H), full),
            pl.BlockSpec((H, H), full),
        ],
        out_specs=[pl.BlockSpec((BN, H), lambda i: (i, 0))] * 3,
        out_shape=[jax.ShapeDtypeStruct((N, H), jnp.float32)] * 3,
    )(h, aggp, aggp, nw1, nw2, nb, ws, wd)


def _node_final_body(h_ref, agg0_ref, agg1_ref, w1_ref, w2_ref, nb_ref,
                     wi_ref, wj_ref, bc_ref,
                     h_ref_o, t1_ref, t2_ref):
    agg = agg0_ref[0] + agg1_ref[0]
    hn = _silu(jnp.dot(h_ref[...], w1_ref[...],
                       preferred_element_type=jnp.float32)
               + jnp.dot(agg, w2_ref[...],
                         preferred_element_type=jnp.float32)
               + nb_ref[...])
    h_ref_o[...] = hn
    bc = bc_ref[...]
    xi = 0.25 * lax.broadcasted_iota(jnp.int32, (1, NPTS), 1).astype(
        jnp.float32)
    mi = 1.0 - bc * (1.0 - xi)
    mj = 1.0 - bc * xi
    ha = jnp.dot(hn, wi_ref[...], preferred_element_type=jnp.float32)
    hb = jnp.dot(hn, wj_ref[...], preferred_element_type=jnp.float32)
    zpad = jnp.zeros((hn.shape[0], 128 - 64 - NPTS), jnp.float32)
    t1_ref[...] = jnp.concatenate([ha, mi, zpad], axis=1)
    t2_ref[...] = jnp.concatenate([hb, mj, zpad], axis=1)


def _node_final(h, aggp, nw1, nw2, nb, wi, wj, bc):
    full = lambda i: (0, 0)
    return pl.pallas_call(
        _node_final_body,
        grid=(N // BN,),
        in_specs=[
            pl.BlockSpec((BN, H), lambda i: (i, 0)),
            pl.BlockSpec((1, BN, H), lambda i: (0, i, 0)),
            pl.BlockSpec((1, BN, H), lambda i: (1, i, 0)),
            pl.BlockSpec((H, H), full),
            pl.BlockSpec((H, H), full),
            pl.BlockSpec((1, H), full),
            pl.BlockSpec((H, 64), full),
            pl.BlockSpec((H, 64), full),
            pl.BlockSpec((BN, 1), lambda i: (i, 0)),
        ],
        out_specs=[
            pl.BlockSpec((BN, H), lambda i: (i, 0)),
            pl.BlockSpec((BN, 128), lambda i: (i, 0)),
            pl.BlockSpec((BN, 128), lambda i: (i, 0)),
        ],
        out_shape=[
            jax.ShapeDtypeStruct((N, H), jnp.float32),
            jax.ShapeDtypeStruct((N, 128), jnp.float32),
            jax.ShapeDtypeStruct((N, 128), jnp.float32),
        ],
    )(h, aggp, aggp, nw1, nw2, nb, wi, wj, bc)


def _field_body(g_ref, w0_ref, b1_ref, w2_ref, b2_ref, w3s_ref, r_ref,
                b3t_ref, f_ref):
    g = g_ref[...]
    base = g[:, :64]
    w0 = w0_ref[...]
    acc = None
    w2b = w2_ref[...].astype(jnp.bfloat16)
    w3b = w3s_ref[...].astype(jnp.bfloat16)
    for k in range(NPTS):
        z = _silu(base + _XI[k] * w0 + b1_ref[...])
        z = _silu(jnp.dot(z.astype(jnp.bfloat16), w2b,
                          preferred_element_type=jnp.float32) + b2_ref[...])
        t = jnp.dot(z.astype(jnp.bfloat16), w3b[64 * k:64 * (k + 1), :],
                    preferred_element_type=jnp.float32)
        acc = t if acc is None else acc + t
    mask15 = jnp.dot(g[:, 64:64 + NPTS], r_ref[...],
                     preferred_element_type=jnp.float32)
    f_ref[...] = (acc + b3t_ref[...]) * mask15


def _field(g, w0, b1, w2, b2, w3s, r, b3t):
    full = lambda i: (0, 0)
    return pl.pallas_call(
        _field_body,
        grid=(E // BE,),
        in_specs=[
            pl.BlockSpec((BE, 128), lambda i: (i, 0)),
            pl.BlockSpec((1, 64), full),
            pl.BlockSpec((1, 64), full),
            pl.BlockSpec((64, 64), full),
            pl.BlockSpec((1, 64), full),
            pl.BlockSpec((NPTS * 64, NPTS * 3), full),
            pl.BlockSpec((NPTS, NPTS * 3), full),
            pl.BlockSpec((1, NPTS * 3), full),
        ],
        out_specs=pl.BlockSpec((BE, NPTS * 3), lambda i: (i, 0)),
        out_shape=jax.ShapeDtypeStruct((E, NPTS * 3), jnp.float32),
    )(g, w0, b1, w2, b2, w3s, r, b3t)


def kernel(x, edge_index, edge_attr, connectivity, bc_disp, prop_I22,
           enc_W1, enc_b1, enc_W2, enc_b2,
           conv0_eW, conv0_eb, conv0_nW, conv0_nb,
           conv1_eW, conv1_eb, conv1_nW, conv1_nb,
           conv2_eW, conv2_eb, conv2_nW, conv2_nb,
           ef_W1, ef_b1, ef_W2, ef_b2, ef_W3, ef_b3):
    convs = [(conv0_eW, conv0_eb, conv0_nW, conv0_nb),
             (conv1_eW, conv1_eb, conv1_nW, conv1_nb),
             (conv2_eW, conv2_eb, conv2_nW, conv2_nb)]

    r2 = lambda v: v.reshape(1, -1)

    def _split(svec, dvec, cut, pt0, pt1, ch):
        ia = jnp.stack([svec[:cut].reshape(NS, pt0 // ch, ch),
                        dvec[:cut].reshape(NS, pt0 // ch, ch)], axis=2)
        ib_ = jnp.stack([svec[cut:].reshape(NS, pt1 // ch, ch),
                         dvec[cut:].reshape(NS, pt1 // ch, ch)], axis=2)
        return ia, ib_

    i2a, i2b = _split(edge_index[0], edge_index[1], C1OFF, PT0, PT1, CCH)
    f2a, f2b = _split(connectivity[:, 0], connectivity[:, 1],
                      F1OFF, PF0, PF1, CHF)

    w3s = jnp.zeros((NPTS * 64, NPTS * 3), jnp.float32)
    rrep = jnp.zeros((NPTS, NPTS * 3), jnp.float32)
    for k in range(NPTS):
        w3s = w3s.at[64 * k:64 * (k + 1), 3 * k:3 * k + 3].set(ef_W3)
        rrep = rrep.at[k, 3 * k:3 * k + 3].set(1.0)
    b3t = jnp.tile(ef_b3, NPTS)[None, :]

    h, a, b = _encode(x, enc_W1, r2(enc_b1), enc_W2, r2(enc_b2),
                      convs[0][0][:H], convs[0][0][H:2 * H])

    for li in range(3):
        eW, eb, nW, nb = convs[li]
        c = _edge_c(edge_attr, eW[2 * H:], r2(eb))
        aggp = _conv_sc(a, b, c, i2a, i2b)
        if li < 2:
            nws, nwd = convs[li + 1][0][:H], convs[li + 1][0][H:2 * H]
            h, a, b = _node_update(h, aggp, nW[:H], nW[H:], r2(nb), nws, nwd)
        else:
            h, t1, t2 = _node_final(h, aggp, nW[:H], nW[H:], r2(nb),
                                    ef_W1[1:1 + H], ef_W1[1 + H:], bc_disp)

    g = _field_sc(t1, t2, f2a, f2b)
    f = _field(g, r2(ef_W1[0]), r2(ef_b1), ef_W2, r2(ef_b2), w3s, rrep, b3t)

    fields = f.reshape(E, NPTS, 3)
    xi = jnp.broadcast_to(
        jnp.asarray(_XI, jnp.float32)[None, :, None], (E, NPTS, 1))
    return h, xi, fields, prop_I22


# speedup vs baseline: 6.2411x; 1.1137x over previous
import functools

import jax
import jax.numpy as jnp
from jax import lax
from jax.experimental import pallas as pl
from jax.experimental.pallas import tpu as pltpu
from jax.experimental.pallas import tpu_sc as plsc

N = 10000
E = 160000
H = 128
NODE_IN = 9
EDGE_DIM = 11
NPTS = 5

BN = 2000
BE = 2000

_XI = [0.0, 0.25, 0.5, 0.75, 1.0]

NC = 2
NS = 16
NPA = 10112
RPSA = NPA // NS

CCH = 40
PT0, PT1 = 5280, 4720
NCH0, NCH1 = PT0 // CCH, PT1 // CCH
C1OFF = NS * PT0

CHF = 40
EA, EB = 83200, 76800
PFA = (2880, 2320)
PFB = (2560, 2240)
BEF = 1600

_SC_MESH = plsc.VectorSubcoreMesh(core_axis_name="c", subcore_axis_name="s")


def _silu(v):
    return v * jax.nn.sigmoid(v)


def _conv_sc(a, b, c, i2a, i2b):
    @functools.partial(
        pl.kernel,
        out_type=jax.ShapeDtypeStruct((NC, NPA, H), jnp.float32),
        mesh=_SC_MESH,
        scratch_types=[
            pltpu.VMEM((2, CCH), jnp.int32),
            pltpu.VMEM((2, CCH), jnp.int32),
            pltpu.VMEM((CCH, H), jnp.float32),
            pltpu.VMEM((CCH, H), jnp.float32),
            pltpu.VMEM((CCH, H), jnp.float32),
            pltpu.VMEM((CCH, H), jnp.float32),
            pltpu.VMEM((CCH, H), jnp.float32),
            pltpu.VMEM((CCH, H), jnp.float32),
            pltpu.VMEM_SHARED((NPA, H), jnp.float32),
            pltpu.SemaphoreType.DMA,
            pltpu.SemaphoreType.DMA,
            pltpu.SemaphoreType.DMA,
            pltpu.SemaphoreType.DMA,
            pltpu.SemaphoreType.DMA,
            pltpu.SemaphoreType.DMA,
        ],
    )
    def k(a_hbm, b_hbm, c_hbm, i2a_hbm, i2b_hbm, out_hbm,
          ib0, ib1, ar0, ar1, br0, br1, cr0, cr1, acc,
          sa0, sa1, sb0, sb1, sc0, sc1):
        cid = lax.axis_index("c")
        sid = lax.axis_index("s")
        zrow = sid * RPSA

        @pl.loop(0, CCH)
        def _(r):
            @pl.loop(0, H, step=16)
            def _(cc):
                ar0[r, pl.ds(cc, 16)] = jnp.zeros((16,), jnp.float32)

        @pl.loop(0, RPSA - RPSA % CCH, step=CCH)
        def _(rr):
            pltpu.sync_copy(ar0, acc.at[pl.ds(zrow + rr, CCH)])
        pltpu.sync_copy(ar0.at[pl.ds(0, RPSA % CCH)],
                        acc.at[pl.ds(zrow + RPSA - RPSA % CCH, RPSA % CCH)])

        plsc.subcore_barrier()

        bufs = ((ib0, ar0, br0, cr0, sa0, sb0, sc0),
                (ib1, ar1, br1, cr1, sa1, sb1, sc1))

        def run(base, nch, i2_hbm):
            def issue(ci, ib, ar, br, cr, sa, sb, sc):
                off = base + ci * CCH
                pltpu.sync_copy(i2_hbm.at[sid, ci], ib)
                pltpu.async_copy(a_hbm.at[ib.at[0]], ar, sa)
                pltpu.async_copy(b_hbm.at[ib.at[1]], br, sb)
                pltpu.async_copy(c_hbm.at[pl.ds(off, CCH)], cr, sc)

            issue(0, *bufs[0])
            issue(1, *bufs[1])

            @pl.loop(0, nch, step=2)
            def _(g):
                for p in range(2):
                    ib, ar, br, cr, sa, sb, sc = bufs[p]
                    ci = g + p
                    pltpu.make_async_copy(a_hbm.at[ib.at[0]], ar, sa).wait()
                    pltpu.make_async_copy(b_hbm.at[ib.at[1]], br, sb).wait()
                    pltpu.make_async_copy(c_hbm.at[pl.ds(base, CCH)], cr,
                                          sc).wait()

                    @pl.loop(0, CCH)
                    def _(r):
                        @pl.loop(0, H, step=16)
                        def _(cc):
                            v = (ar[r, pl.ds(cc, 16)] + br[r, pl.ds(cc, 16)]
                                 + cr[r, pl.ds(cc, 16)])
                            ar[r, pl.ds(cc, 16)] = v / (1.0 + jnp.exp(-v))

                    pltpu.sync_copy(ar, acc.at[ib.at[1]], add=True)

                    @pl.when(ci + 2 < nch)
                    def _():
                        issue(ci + 2, *bufs[p])

        @pl.when(cid == 0)
        def _():
            run(sid * PT0, NCH0, i2a_hbm)

        @pl.when(cid == 1)
        def _():
            run(C1OFF + sid * PT1, NCH1, i2b_hbm)

        plsc.subcore_barrier()
        pltpu.sync_copy(acc.at[pl.ds(zrow, RPSA)],
                        out_hbm.at[cid, pl.ds(zrow, RPSA)])

    return k(a, b, c, i2a, i2b)


def _field_sc(e_tot, pf, t1, t2, f2a, f2b):
    @functools.partial(
        pl.kernel,
        out_type=jax.ShapeDtypeStruct((e_tot, 128), jnp.float32),
        mesh=_SC_MESH,
        scratch_types=[
            pltpu.VMEM((2, CHF), jnp.int32),
            pltpu.VMEM((2, CHF), jnp.int32),
            pltpu.VMEM((CHF, 128), jnp.float32),
            pltpu.VMEM((CHF, 128), jnp.float32),
            pltpu.VMEM((CHF, 128), jnp.float32),
            pltpu.VMEM((CHF, 128), jnp.float32),
            pltpu.VMEM((CHF, 128), jnp.float32),
            pltpu.VMEM((CHF, 128), jnp.float32),
            pltpu.SemaphoreType.DMA,
            pltpu.SemaphoreType.DMA,
            pltpu.SemaphoreType.DMA,
            pltpu.SemaphoreType.DMA,
            pltpu.SemaphoreType.DMA,
            pltpu.SemaphoreType.DMA,
        ],
    )
    def k(t1_hbm, t2_hbm, f2a_hbm, f2b_hbm, out_hbm,
          ib0, ib1, g10, g11, g20, g21, ob0, ob1,
          s10, s11, s20, s21, so0, so1):
        cid = lax.axis_index("c")
        sid = lax.axis_index("s")

        bufs = ((ib0, g10, g20, ob0, s10, s20, so0),
                (ib1, g11, g21, ob1, s11, s21, so1))

        def run(base, nf, i2_hbm):
            def issue(ci, ib, g1, g2, ob, s1, s2, so):
                off = base + ci * CHF
                pltpu.sync_copy(i2_hbm.at[sid, ci], ib)
                pltpu.async_copy(t1_hbm.at[ib.at[0]], g1, s1)
                pltpu.async_copy(t2_hbm.at[ib.at[1]], g2, s2)

            issue(0, *bufs[0])
            issue(1, *bufs[1])

            @pl.loop(0, nf, step=2)
            def _(g):
                for p in range(2):
                    ib, g1, g2, ob, s1, s2, so = bufs[p]
                    ci = g + p
                    pltpu.make_async_copy(t1_hbm.at[ib.at[0]], g1, s1).wait()
                    pltpu.make_async_copy(t2_hbm.at[ib.at[1]], g2, s2).wait()

                    @pl.when(ci >= 2)
                    def _():
                        pltpu.make_async_copy(
                            ob, out_hbm.at[pl.ds(base, CHF)], so).wait()

                    @pl.loop(0, CHF)
                    def _(r):
                        @pl.loop(0, 64, step=16)
                        def _(cc):
                            ob[r, pl.ds(cc, 16)] = (g1[r, pl.ds(cc, 16)]
                                                    + g2[r, pl.ds(cc, 16)])
                        ob[r, pl.ds(64, 16)] = (g1[r, pl.ds(64, 16)]
                                                * g2[r, pl.ds(64, 16)])

                    pltpu.async_copy(
                        ob, out_hbm.at[pl.ds(base + ci * CHF, CHF)], so)

                    @pl.when(ci + 2 < nf)
                    def _():
                        issue(ci + 2, *bufs[p])

            pltpu.make_async_copy(ob0, out_hbm.at[pl.ds(base, CHF)],
                                  so0).wait()
            pltpu.make_async_copy(ob1, out_hbm.at[pl.ds(base, CHF)],
                                  so1).wait()

        @pl.when(cid == 0)
        def _():
            run(sid * pf[0], pf[0] // CHF, f2a_hbm)

        @pl.when(cid == 1)
        def _():
            run(NS * pf[0] + sid * pf[1], pf[1] // CHF, f2b_hbm)

    return k(t1, t2, f2a, f2b)


def _enc_body(x_ref, w1_ref, b1_ref, w2_ref, b2_ref, ws_ref, wd_ref,
              h_ref, a_ref, b_ref):
    h1 = _silu(jnp.dot(x_ref[...], w1_ref[...],
                       preferred_element_type=jnp.float32) + b1_ref[...])
    h2 = _silu(jnp.dot(h1, w2_ref[...],
                       preferred_element_type=jnp.float32) + b2_ref[...])
    h_ref[...] = h2
    a_ref[...] = jnp.dot(h2, ws_ref[...], preferred_element_type=jnp.float32)
    b_ref[...] = jnp.dot(h2, wd_ref[...], preferred_element_type=jnp.float32)


def _encode(x, w1, b1, w2, b2, ws, wd):
    full = lambda i: (0, 0)
    return pl.pallas_call(
        _enc_body,
        grid=(N // BN,),
        in_specs=[
            pl.BlockSpec((BN, NODE_IN), lambda i: (i, 0)),
            pl.BlockSpec((NODE_IN, H), full),
            pl.BlockSpec((1, H), full),
            pl.BlockSpec((H, H), full),
            pl.BlockSpec((1, H), full),
            pl.BlockSpec((H, H), full),
            pl.BlockSpec((H, H), full),
        ],
        out_specs=[pl.BlockSpec((BN, H), lambda i: (i, 0))] * 3,
        out_shape=[jax.ShapeDtypeStruct((N, H), jnp.float32)] * 3,
    )(x, w1, b1, w2, b2, ws, wd)


def _edge_c_body(ea_ref, wa_ref, eb_ref, c_ref):
    c_ref[...] = jnp.dot(ea_ref[...], wa_ref[...],
                         preferred_element_type=jnp.float32) + eb_ref[...]


def _edge_c(ea, wa, eb):
    return pl.pallas_call(
        _edge_c_body,
        grid=(E // BE,),
        in_specs=[
            pl.BlockSpec((BE, EDGE_DIM), lambda i: (i, 0)),
            pl.BlockSpec((EDGE_DIM, H), lambda i: (0, 0)),
            pl.BlockSpec((1, H), lambda i: (0, 0)),
        ],
        out_specs=pl.BlockSpec((BE, H), lambda i: (i, 0)),
        out_shape=jax.ShapeDtypeStruct((E, H), jnp.float32),
    )(ea, wa, eb)


def _node_upd_body(h_ref, agg0_ref, agg1_ref, w1_ref, w2_ref, nb_ref,
                   ws_ref, wd_ref, h_ref_o, a_ref, b_ref):
    agg = agg0_ref[0] + agg1_ref[0]
    hn = _silu(jnp.dot(h_ref[...], w1_ref[...],
                       preferred_element_type=jnp.float32)
               + jnp.dot(agg, w2_ref[...],
                         preferred_element_type=jnp.float32)
               + nb_ref[...])
    h_ref_o[...] = hn
    a_ref[...] = jnp.dot(hn, ws_ref[...], preferred_element_type=jnp.float32)
    b_ref[...] = jnp.dot(hn, wd_ref[...], preferred_element_type=jnp.float32)


def _node_update(h, aggp, nw1, nw2, nb, ws, wd):
    full = lambda i: (0, 0)
    return pl.pallas_call(
        _node_upd_body,
        grid=(N // BN,),
        in_specs=[
            pl.BlockSpec((BN, H), lambda i: (i, 0)),
            pl.BlockSpec((1, BN, H), lambda i: (0, i, 0)),
            pl.BlockSpec((1, BN, H), lambda i: (1, i, 0)),
            pl.BlockSpec((H, H), full),
            pl.BlockSpec((H, H), full),
            pl.BlockSpec((1, H), full),
            pl.BlockSpec((H, H), full),
            pl.BlockSpec((H, H), full),
        ],
        out_specs=[pl.BlockSpec((BN, H), lambda i: (i, 0))] * 3,
        out_shape=[jax.ShapeDtypeStruct((N, H), jnp.float32)] * 3,
    )(h, aggp, aggp, nw1, nw2, nb, ws, wd)


def _node_final_body(h_ref, agg0_ref, agg1_ref, w1_ref, w2_ref, nb_ref,
                     wi_ref, wj_ref, bc_ref,
                     h_ref_o, t1_ref, t2_ref):
    agg = agg0_ref[0] + agg1_ref[0]
    hn = _silu(jnp.dot(h_ref[...], w1_ref[...],
                       preferred_element_type=jnp.float32)
               + jnp.dot(agg, w2_ref[...],
                         preferred_element_type=jnp.float32)
               + nb_ref[...])
    h_ref_o[...] = hn
    bc = bc_ref[...]
    xi = 0.25 * lax.broadcasted_iota(jnp.int32, (1, NPTS), 1).astype(
        jnp.float32)
    mi = 1.0 - bc * (1.0 - xi)
    mj = 1.0 - bc * xi
    ha = jnp.dot(hn, wi_ref[...], preferred_element_type=jnp.float32)
    hb = jnp.dot(hn, wj_ref[...], preferred_element_type=jnp.float32)
    zpad = jnp.zeros((hn.shape[0], 128 - 64 - NPTS), jnp.float32)
    t1_ref[...] = jnp.concatenate([ha, mi, zpad], axis=1)
    t2_ref[...] = jnp.concatenate([hb, mj, zpad], axis=1)


def _node_final(h, aggp, nw1, nw2, nb, wi, wj, bc):
    full = lambda i: (0, 0)
    return pl.pallas_call(
        _node_final_body,
        grid=(N // BN,),
        in_specs=[
            pl.BlockSpec((BN, H), lambda i: (i, 0)),
            pl.BlockSpec((1, BN, H), lambda i: (0, i, 0)),
            pl.BlockSpec((1, BN, H), lambda i: (1, i, 0)),
            pl.BlockSpec((H, H), full),
            pl.BlockSpec((H, H), full),
            pl.BlockSpec((1, H), full),
            pl.BlockSpec((H, 64), full),
            pl.BlockSpec((H, 64), full),
            pl.BlockSpec((BN, 1), lambda i: (i, 0)),
        ],
        out_specs=[
            pl.BlockSpec((BN, H), lambda i: (i, 0)),
            pl.BlockSpec((BN, 128), lambda i: (i, 0)),
            pl.BlockSpec((BN, 128), lambda i: (i, 0)),
        ],
        out_shape=[
            jax.ShapeDtypeStruct((N, H), jnp.float32),
            jax.ShapeDtypeStruct((N, 128), jnp.float32),
            jax.ShapeDtypeStruct((N, 128), jnp.float32),
        ],
    )(h, aggp, aggp, nw1, nw2, nb, wi, wj, bc)


def _field_body(g_ref, w0_ref, b1_ref, w2_ref, b2_ref, w3s_ref, r_ref,
                b3t_ref, f_ref):
    g = g_ref[...]
    base = g[:, :64]
    w0 = w0_ref[...]
    acc = None
    for k in range(NPTS):
        z = _silu(base + _XI[k] * w0 + b1_ref[...])
        z = _silu(jnp.dot(z, w2_ref[...],
                          preferred_element_type=jnp.float32) + b2_ref[...])
        t = jnp.dot(z, w3s_ref[64 * k:64 * (k + 1), :],
                    preferred_element_type=jnp.float32)
        acc = t if acc is None else acc + t
    mask15 = jnp.dot(g[:, 64:64 + NPTS], r_ref[...],
                     preferred_element_type=jnp.float32)
    f_ref[...] = (acc + b3t_ref[...]) * mask15


def _field(g, w0, b1, w2, b2, w3s, r, b3t):
    e_tot = g.shape[0]
    full = lambda i: (0, 0)
    return pl.pallas_call(
        _field_body,
        grid=(e_tot // BEF,),
        in_specs=[
            pl.BlockSpec((BEF, 128), lambda i: (i, 0)),
            pl.BlockSpec((1, 64), full),
            pl.BlockSpec((1, 64), full),
            pl.BlockSpec((64, 64), full),
            pl.BlockSpec((1, 64), full),
            pl.BlockSpec((NPTS * 64, NPTS * 3), full),
            pl.BlockSpec((NPTS, NPTS * 3), full),
            pl.BlockSpec((1, NPTS * 3), full),
        ],
        out_specs=pl.BlockSpec((BEF, NPTS * 3), lambda i: (i, 0)),
        out_shape=jax.ShapeDtypeStruct((e_tot, NPTS * 3), jnp.float32),
    )(g, w0, b1, w2, b2, w3s, r, b3t)


def kernel(x, edge_index, edge_attr, connectivity, bc_disp, prop_I22,
           enc_W1, enc_b1, enc_W2, enc_b2,
           conv0_eW, conv0_eb, conv0_nW, conv0_nb,
           conv1_eW, conv1_eb, conv1_nW, conv1_nb,
           conv2_eW, conv2_eb, conv2_nW, conv2_nb,
           ef_W1, ef_b1, ef_W2, ef_b2, ef_W3, ef_b3):
    convs = [(conv0_eW, conv0_eb, conv0_nW, conv0_nb),
             (conv1_eW, conv1_eb, conv1_nW, conv1_nb),
             (conv2_eW, conv2_eb, conv2_nW, conv2_nb)]

    r2 = lambda v: v.reshape(1, -1)

    def _split(svec, dvec, cut, pt0, pt1, ch):
        ia = jnp.stack([svec[:cut].reshape(NS, pt0 // ch, ch),
                        dvec[:cut].reshape(NS, pt0 // ch, ch)], axis=2)
        ib_ = jnp.stack([svec[cut:].reshape(NS, pt1 // ch, ch),
                         dvec[cut:].reshape(NS, pt1 // ch, ch)], axis=2)
        return ia, ib_

    i2a, i2b = _split(edge_index[0], edge_index[1], C1OFF, PT0, PT1, CCH)
    n1, n2 = connectivity[:, 0], connectivity[:, 1]
    fA0, fA1 = _split(n1[:EA], n2[:EA], NS * PFA[0], PFA[0], PFA[1], CHF)
    fB0, fB1 = _split(n1[EA:], n2[EA:], NS * PFB[0], PFB[0], PFB[1], CHF)

    w3s = jnp.zeros((NPTS * 64, NPTS * 3), jnp.float32)
    rrep = jnp.zeros((NPTS, NPTS * 3), jnp.float32)
    for k in range(NPTS):
        w3s = w3s.at[64 * k:64 * (k + 1), 3 * k:3 * k + 3].set(ef_W3)
        rrep = rrep.at[k, 3 * k:3 * k + 3].set(1.0)
    b3t = jnp.tile(ef_b3, NPTS)[None, :]

    h, a, b = _encode(x, enc_W1, r2(enc_b1), enc_W2, r2(enc_b2),
                      convs[0][0][:H], convs[0][0][H:2 * H])

    for li in range(3):
        eW, eb, nW, nb = convs[li]
        c = _edge_c(edge_attr, eW[2 * H:], r2(eb))
        aggp = _conv_sc(a, b, c, i2a, i2b)
        if li < 2:
            nws, nwd = convs[li + 1][0][:H], convs[li + 1][0][H:2 * H]
            h, a, b = _node_update(h, aggp, nW[:H], nW[H:], r2(nb), nws, nwd)
        else:
            h, t1, t2 = _node_final(h, aggp, nW[:H], nW[H:], r2(nb),
                                    ef_W1[1:1 + H], ef_W1[1 + H:], bc_disp)

    gA = _field_sc(EA, PFA, t1, t2, fA0, fA1)
    fa = _field(gA, r2(ef_W1[0]), r2(ef_b1), ef_W2, r2(ef_b2), w3s, rrep, b3t)
    gB = _field_sc(EB, PFB, t1, t2, fB0, fB1)
    fb = _field(gB, r2(ef_W1[0]), r2(ef_b1), ef_W2, r2(ef_b2), w3s, rrep, b3t)
    f = jnp.concatenate([fa, fb], axis=0)

    fields = f.reshape(E, NPTS, 3)
    xi = jnp.broadcast_to(
        jnp.asarray(_XI, jnp.float32)[None, :, None], (E, NPTS, 1))
    return h, xi, fields, prop_I22


# speedup vs baseline: 6.5284x; 1.0460x over previous
import functools

import jax
import jax.numpy as jnp
from jax import lax
from jax.experimental import pallas as pl
from jax.experimental.pallas import tpu as pltpu
from jax.experimental.pallas import tpu_sc as plsc

N = 10000
E = 160000
H = 128
NODE_IN = 9
EDGE_DIM = 11
NPTS = 5

BN = 2000
BE = 8000

_XI = [0.0, 0.25, 0.5, 0.75, 1.0]

NC = 2
NS = 16
NPA = 10112
RPSA = NPA // NS

CCH = 40
PT0, PT1 = 5120, 4880
NCH0, NCH1 = PT0 // CCH, PT1 // CCH
C1OFF = NS * PT0

CHF = 40
EA, EB = 83200, 76800
PFA = (2880, 2320)
PFB = (2480, 2320)
BEF = 1600

_SC_MESH = plsc.VectorSubcoreMesh(core_axis_name="c", subcore_axis_name="s")


def _silu(v):
    return v * jax.nn.sigmoid(v)


def _conv_sc(a, b, c, i2a, i2b):
    @functools.partial(
        pl.kernel,
        out_type=jax.ShapeDtypeStruct((NC, NPA, H), jnp.float32),
        mesh=_SC_MESH,
        scratch_types=[
            pltpu.VMEM((1, CCH), jnp.int32),
            pltpu.VMEM((1, CCH), jnp.int32),
            pltpu.VMEM((CCH,), jnp.int32),
            pltpu.VMEM((CCH,), jnp.int32),
            pltpu.VMEM((CCH,), jnp.int32),
            pltpu.VMEM((CCH,), jnp.int32),
            pltpu.VMEM((CCH, H), jnp.float32),
            pltpu.VMEM((CCH, H), jnp.float32),
            pltpu.VMEM((CCH, H), jnp.float32),
            pltpu.VMEM((CCH, H), jnp.float32),
            pltpu.VMEM((CCH, H), jnp.float32),
            pltpu.VMEM((CCH, H), jnp.float32),
            pltpu.VMEM_SHARED((NPA, H), jnp.float32),
            pltpu.SemaphoreType.DMA,
            pltpu.SemaphoreType.DMA,
            pltpu.SemaphoreType.DMA,
            pltpu.SemaphoreType.DMA,
            pltpu.SemaphoreType.DMA,
            pltpu.SemaphoreType.DMA,
        ],
    )
    def k(a_hbm, b_hbm, c_hbm, i2a_hbm, i2b_hbm, out_hbm,
          ib0, ib1, is0, is1, id0, id1, ar0, ar1, br0, br1, cr0, cr1, acc,
          sa0, sa1, sb0, sb1, sc0, sc1):
        cid = lax.axis_index("c")
        sid = lax.axis_index("s")
        zrow = sid * RPSA

        @pl.loop(0, CCH)
        def _(r):
            @pl.loop(0, H, step=16)
            def _(cc):
                ar0[r, pl.ds(cc, 16)] = jnp.zeros((16,), jnp.float32)

        @pl.loop(0, RPSA - RPSA % CCH, step=CCH)
        def _(rr):
            pltpu.sync_copy(ar0, acc.at[pl.ds(zrow + rr, CCH)])
        pltpu.sync_copy(ar0.at[pl.ds(0, RPSA % CCH)],
                        acc.at[pl.ds(zrow + RPSA - RPSA % CCH, RPSA % CCH)])

        plsc.subcore_barrier()

        bufs = ((ib0, is0, id0, ar0, br0, cr0, sa0, sb0, sc0),
                (ib1, is1, id1, ar1, br1, cr1, sa1, sb1, sc1))

        def run(base, nch, i2_hbm):
            def issue(ci, ib, isb, idb, ar, br, cr, sa, sb, sc):
                off = base + ci * CCH
                pltpu.sync_copy(i2_hbm.at[sid, ci], ib)
                for o in (0, 16, CCH - 16):
                    v = ib[0, pl.ds(o, 16)]
                    isb[pl.ds(o, 16)] = jnp.bitwise_and(v, 65535)
                    idb[pl.ds(o, 16)] = jnp.right_shift(v, 16)
                pltpu.async_copy(a_hbm.at[isb], ar, sa)
                pltpu.async_copy(b_hbm.at[idb], br, sb)
                pltpu.async_copy(c_hbm.at[pl.ds(off, CCH)], cr, sc)

            issue(0, *bufs[0])
            issue(1, *bufs[1])

            @pl.loop(0, nch, step=2)
            def _(g):
                for p in range(2):
                    ib, isb, idb, ar, br, cr, sa, sb, sc = bufs[p]
                    ci = g + p
                    pltpu.make_async_copy(a_hbm.at[isb], ar, sa).wait()
                    pltpu.make_async_copy(b_hbm.at[idb], br, sb).wait()
                    pltpu.make_async_copy(c_hbm.at[pl.ds(base, CCH)], cr,
                                          sc).wait()

                    @pl.loop(0, CCH)
                    def _(r):
                        @pl.loop(0, H, step=16)
                        def _(cc):
                            v = (ar[r, pl.ds(cc, 16)] + br[r, pl.ds(cc, 16)]
                                 + cr[r, pl.ds(cc, 16)])
                            ar[r, pl.ds(cc, 16)] = v / (1.0 + jnp.exp(-v))

                    pltpu.sync_copy(ar, acc.at[idb], add=True)

                    @pl.when(ci + 2 < nch)
                    def _():
                        issue(ci + 2, *bufs[p])

        @pl.when(cid == 0)
        def _():
            run(sid * PT0, NCH0, i2a_hbm)

        @pl.when(cid == 1)
        def _():
            run(C1OFF + sid * PT1, NCH1, i2b_hbm)

        plsc.subcore_barrier()
        pltpu.sync_copy(acc.at[pl.ds(zrow, RPSA)],
                        out_hbm.at[cid, pl.ds(zrow, RPSA)])

    return k(a, b, c, i2a, i2b)


def _field_sc(e_tot, pf, t1, t2, f2a, f2b):
    @functools.partial(
        pl.kernel,
        out_type=jax.ShapeDtypeStruct((e_tot, 128), jnp.float32),
        mesh=_SC_MESH,
        scratch_types=[
            pltpu.VMEM((1, CHF), jnp.int32),
            pltpu.VMEM((1, CHF), jnp.int32),
            pltpu.VMEM((CHF,), jnp.int32),
            pltpu.VMEM((CHF,), jnp.int32),
            pltpu.VMEM((CHF,), jnp.int32),
            pltpu.VMEM((CHF,), jnp.int32),
            pltpu.VMEM((CHF, 128), jnp.float32),
            pltpu.VMEM((CHF, 128), jnp.float32),
            pltpu.VMEM((CHF, 128), jnp.float32),
            pltpu.VMEM((CHF, 128), jnp.float32),
            pltpu.VMEM((CHF, 128), jnp.float32),
            pltpu.VMEM((CHF, 128), jnp.float32),
            pltpu.SemaphoreType.DMA,
            pltpu.SemaphoreType.DMA,
            pltpu.SemaphoreType.DMA,
            pltpu.SemaphoreType.DMA,
            pltpu.SemaphoreType.DMA,
            pltpu.SemaphoreType.DMA,
        ],
    )
    def k(t1_hbm, t2_hbm, f2a_hbm, f2b_hbm, out_hbm,
          ib0, ib1, is0, is1, id0, id1, g10, g11, g20, g21, ob0, ob1,
          s10, s11, s20, s21, so0, so1):
        cid = lax.axis_index("c")
        sid = lax.axis_index("s")

        bufs = ((ib0, is0, id0, g10, g20, ob0, s10, s20, so0),
                (ib1, is1, id1, g11, g21, ob1, s11, s21, so1))

        def run(base, nf, i2_hbm):
            def issue(ci, ib, isb, idb, g1, g2, ob, s1, s2, so):
                pltpu.sync_copy(i2_hbm.at[sid, ci], ib)
                for o in (0, 16, CHF - 16):
                    v = ib[0, pl.ds(o, 16)]
                    isb[pl.ds(o, 16)] = jnp.bitwise_and(v, 65535)
                    idb[pl.ds(o, 16)] = jnp.right_shift(v, 16)
                pltpu.async_copy(t1_hbm.at[isb], g1, s1)
                pltpu.async_copy(t2_hbm.at[idb], g2, s2)

            issue(0, *bufs[0])
            issue(1, *bufs[1])

            @pl.loop(0, nf, step=2)
            def _(g):
                for p in range(2):
                    ib, isb, idb, g1, g2, ob, s1, s2, so = bufs[p]
                    ci = g + p
                    pltpu.make_async_copy(t1_hbm.at[isb], g1, s1).wait()
                    pltpu.make_async_copy(t2_hbm.at[idb], g2, s2).wait()

                    @pl.when(ci >= 2)
                    def _():
                        pltpu.make_async_copy(
                            ob, out_hbm.at[pl.ds(base, CHF)], so).wait()

                    @pl.loop(0, CHF)
                    def _(r):
                        @pl.loop(0, 64, step=16)
                        def _(cc):
                            ob[r, pl.ds(cc, 16)] = (g1[r, pl.ds(cc, 16)]
                                                    + g2[r, pl.ds(cc, 16)])
                        ob[r, pl.ds(64, 16)] = (g1[r, pl.ds(64, 16)]
                                                * g2[r, pl.ds(64, 16)])

                    pltpu.async_copy(
                        ob, out_hbm.at[pl.ds(base + ci * CHF, CHF)], so)

                    @pl.when(ci + 2 < nf)
                    def _():
                        issue(ci + 2, *bufs[p])

            pltpu.make_async_copy(ob0, out_hbm.at[pl.ds(base, CHF)],
                                  so0).wait()
            pltpu.make_async_copy(ob1, out_hbm.at[pl.ds(base, CHF)],
                                  so1).wait()

        @pl.when(cid == 0)
        def _():
            run(sid * pf[0], pf[0] // CHF, f2a_hbm)

        @pl.when(cid == 1)
        def _():
            run(NS * pf[0] + sid * pf[1], pf[1] // CHF, f2b_hbm)

    return k(t1, t2, f2a, f2b)


def _enc_body(x_ref, w1_ref, b1_ref, w2_ref, b2_ref, ws_ref, wd_ref,
              h_ref, a_ref, b_ref):
    h1 = _silu(jnp.dot(x_ref[...], w1_ref[...],
                       preferred_element_type=jnp.float32) + b1_ref[...])
    h2 = _silu(jnp.dot(h1, w2_ref[...],
                       preferred_element_type=jnp.float32) + b2_ref[...])
    h_ref[...] = h2
    a_ref[...] = jnp.dot(h2, ws_ref[...], preferred_element_type=jnp.float32)
    b_ref[...] = jnp.dot(h2, wd_ref[...], preferred_element_type=jnp.float32)


def _encode(x, w1, b1, w2, b2, ws, wd):
    full = lambda i: (0, 0)
    return pl.pallas_call(
        _enc_body,
        grid=(N // BN,),
        in_specs=[
            pl.BlockSpec((BN, NODE_IN), lambda i: (i, 0)),
            pl.BlockSpec((NODE_IN, H), full),
            pl.BlockSpec((1, H), full),
            pl.BlockSpec((H, H), full),
            pl.BlockSpec((1, H), full),
            pl.BlockSpec((H, H), full),
            pl.BlockSpec((H, H), full),
        ],
        out_specs=[pl.BlockSpec((BN, H), lambda i: (i, 0))] * 3,
        out_shape=[jax.ShapeDtypeStruct((N, H), jnp.float32)] * 3,
    )(x, w1, b1, w2, b2, ws, wd)


def _edge_c_body(ea_ref, wa_ref, eb_ref, c_ref):
    c_ref[...] = jnp.dot(ea_ref[...], wa_ref[...],
                         preferred_element_type=jnp.float32) + eb_ref[...]


def _edge_c(ea, wa, eb):
    return pl.pallas_call(
        _edge_c_body,
        grid=(E // BE,),
        in_specs=[
            pl.BlockSpec((BE, EDGE_DIM), lambda i: (i, 0)),
            pl.BlockSpec((EDGE_DIM, H), lambda i: (0, 0)),
            pl.BlockSpec((1, H), lambda i: (0, 0)),
        ],
        out_specs=pl.BlockSpec((BE, H), lambda i: (i, 0)),
        out_shape=jax.ShapeDtypeStruct((E, H), jnp.float32),
    )(ea, wa, eb)


def _node_upd_body(h_ref, agg0_ref, agg1_ref, w1_ref, w2_ref, nb_ref,
                   ws_ref, wd_ref, h_ref_o, a_ref, b_ref):
    agg = agg0_ref[0] + agg1_ref[0]
    hn = _silu(jnp.dot(h_ref[...], w1_ref[...],
                       preferred_element_type=jnp.float32)
               + jnp.dot(agg, w2_ref[...],
                         preferred_element_type=jnp.float32)
               + nb_ref[...])
    h_ref_o[...] = hn
    a_ref[...] = jnp.dot(hn, ws_ref[...], preferred_element_type=jnp.float32)
    b_ref[...] = jnp.dot(hn, wd_ref[...], preferred_element_type=jnp.float32)


def _node_update(h, aggp, nw1, nw2, nb, ws, wd):
    full = lambda i: (0, 0)
    return pl.pallas_call(
        _node_upd_body,
        grid=(N // BN,),
        in_specs=[
            pl.BlockSpec((BN, H), lambda i: (i, 0)),
            pl.BlockSpec((1, BN, H), lambda i: (0, i, 0)),
            pl.BlockSpec((1, BN, H), lambda i: (1, i, 0)),
            pl.BlockSpec((H, H), full),
            pl.BlockSpec((H, H), full),
            pl.BlockSpec((1, H), full),
            pl.BlockSpec((H, H), full),
            pl.BlockSpec((H, H), full),
        ],
        out_specs=[pl.BlockSpec((BN, H), lambda i: (i, 0))] * 3,
        out_shape=[jax.ShapeDtypeStruct((N, H), jnp.float32)] * 3,
    )(h, aggp, aggp, nw1, nw2, nb, ws, wd)


def _node_final_body(h_ref, agg0_ref, agg1_ref, w1_ref, w2_ref, nb_ref,
                     wi_ref, wj_ref, bc_ref,
                     h_ref_o, t1_ref, t2_ref):
    agg = agg0_ref[0] + agg1_ref[0]
    hn = _silu(jnp.dot(h_ref[...], w1_ref[...],
                       preferred_element_type=jnp.float32)
               + jnp.dot(agg, w2_ref[...],
                         preferred_element_type=jnp.float32)
               + nb_ref[...])
    h_ref_o[...] = hn
    bc = bc_ref[...]
    xi = 0.25 * lax.broadcasted_iota(jnp.int32, (1, NPTS), 1).astype(
        jnp.float32)
    mi = 1.0 - bc * (1.0 - xi)
    mj = 1.0 - bc * xi
    ha = jnp.dot(hn, wi_ref[...], preferred_element_type=jnp.float32)
    hb = jnp.dot(hn, wj_ref[...], preferred_element_type=jnp.float32)
    zpad = jnp.zeros((hn.shape[0], 128 - 64 - NPTS), jnp.float32)
    t1_ref[...] = jnp.concatenate([ha, mi, zpad], axis=1)
    t2_ref[...] = jnp.concatenate([hb, mj, zpad], axis=1)


def _node_final(h, aggp, nw1, nw2, nb, wi, wj, bc):
    full = lambda i: (0, 0)
    return pl.pallas_call(
        _node_final_body,
        grid=(N // BN,),
        in_specs=[
            pl.BlockSpec((BN, H), lambda i: (i, 0)),
            pl.BlockSpec((1, BN, H), lambda i: (0, i, 0)),
            pl.BlockSpec((1, BN, H), lambda i: (1, i, 0)),
            pl.BlockSpec((H, H), full),
            pl.BlockSpec((H, H), full),
            pl.BlockSpec((1, H), full),
            pl.BlockSpec((H, 64), full),
            pl.BlockSpec((H, 64), full),
            pl.BlockSpec((BN, 1), lambda i: (i, 0)),
        ],
        out_specs=[
            pl.BlockSpec((BN, H), lambda i: (i, 0)),
            pl.BlockSpec((BN, 128), lambda i: (i, 0)),
            pl.BlockSpec((BN, 128), lambda i: (i, 0)),
        ],
        out_shape=[
            jax.ShapeDtypeStruct((N, H), jnp.float32),
            jax.ShapeDtypeStruct((N, 128), jnp.float32),
            jax.ShapeDtypeStruct((N, 128), jnp.float32),
        ],
    )(h, aggp, aggp, nw1, nw2, nb, wi, wj, bc)


def _field_body(g_ref, w0_ref, b1_ref, w2_ref, b2_ref, w3s_ref, r_ref,
                b3t_ref, f_ref):
    g = g_ref[...]
    base = g[:, :64]
    w0 = w0_ref[...]
    acc = None
    for k in range(NPTS):
        z = _silu(base + _XI[k] * w0 + b1_ref[...])
        z = _silu(jnp.dot(z, w2_ref[...],
                          preferred_element_type=jnp.float32) + b2_ref[...])
        t = jnp.dot(z, w3s_ref[64 * k:64 * (k + 1), :],
                    preferred_element_type=jnp.float32)
        acc = t if acc is None else acc + t
    mask15 = jnp.dot(g[:, 64:64 + NPTS], r_ref[...],
                     preferred_element_type=jnp.float32)
    f_ref[...] = (acc + b3t_ref[...]) * mask15


def _field(g, w0, b1, w2, b2, w3s, r, b3t):
    e_tot = g.shape[0]
    full = lambda i: (0, 0)
    return pl.pallas_call(
        _field_body,
        grid=(e_tot // BEF,),
        in_specs=[
            pl.BlockSpec((BEF, 128), lambda i: (i, 0)),
            pl.BlockSpec((1, 64), full),
            pl.BlockSpec((1, 64), full),
            pl.BlockSpec((64, 64), full),
            pl.BlockSpec((1, 64), full),
            pl.BlockSpec((NPTS * 64, NPTS * 3), full),
            pl.BlockSpec((NPTS, NPTS * 3), full),
            pl.BlockSpec((1, NPTS * 3), full),
        ],
        out_specs=pl.BlockSpec((BEF, NPTS * 3), lambda i: (i, 0)),
        out_shape=jax.ShapeDtypeStruct((e_tot, NPTS * 3), jnp.float32),
    )(g, w0, b1, w2, b2, w3s, r, b3t)


def kernel(x, edge_index, edge_attr, connectivity, bc_disp, prop_I22,
           enc_W1, enc_b1, enc_W2, enc_b2,
           conv0_eW, conv0_eb, conv0_nW, conv0_nb,
           conv1_eW, conv1_eb, conv1_nW, conv1_nb,
           conv2_eW, conv2_eb, conv2_nW, conv2_nb,
           ef_W1, ef_b1, ef_W2, ef_b2, ef_W3, ef_b3):
    convs = [(conv0_eW, conv0_eb, conv0_nW, conv0_nb),
             (conv1_eW, conv1_eb, conv1_nW, conv1_nb),
             (conv2_eW, conv2_eb, conv2_nW, conv2_nb)]

    r2 = lambda v: v.reshape(1, -1)

    def _split(svec, dvec, cut, pt0, pt1, ch):
        pk = svec + (dvec << 16)
        ia = pk[:cut].reshape(NS, pt0 // ch, 1, ch)
        ib_ = pk[cut:].reshape(NS, pt1 // ch, 1, ch)
        return ia, ib_

    i2a, i2b = _split(edge_index[0], edge_index[1], C1OFF, PT0, PT1, CCH)
    n1, n2 = connectivity[:, 0], connectivity[:, 1]
    fA0, fA1 = _split(n1[:EA], n2[:EA], NS * PFA[0], PFA[0], PFA[1], CHF)
    fB0, fB1 = _split(n1[EA:], n2[EA:], NS * PFB[0], PFB[0], PFB[1], CHF)

    w3s = jnp.zeros((NPTS * 64, NPTS * 3), jnp.float32)
    rrep = jnp.zeros((NPTS, NPTS * 3), jnp.float32)
    for k in range(NPTS):
        w3s = w3s.at[64 * k:64 * (k + 1), 3 * k:3 * k + 3].set(ef_W3)
        rrep = rrep.at[k, 3 * k:3 * k + 3].set(1.0)
    b3t = jnp.tile(ef_b3, NPTS)[None, :]

    h, a, b = _encode(x, enc_W1, r2(enc_b1), enc_W2, r2(enc_b2),
                      convs[0][0][:H], convs[0][0][H:2 * H])

    for li in range(3):
        eW, eb, nW, nb = convs[li]
        c = _edge_c(edge_attr, eW[2 * H:], r2(eb))
        aggp = _conv_sc(a, b, c, i2a, i2b)
        if li < 2:
            nws, nwd = convs[li + 1][0][:H], convs[li + 1][0][H:2 * H]
            h, a, b = _node_update(h, aggp, nW[:H], nW[H:], r2(nb), nws, nwd)
        else:
            h, t1, t2 = _node_final(h, aggp, nW[:H], nW[H:], r2(nb),
                                    ef_W1[1:1 + H], ef_W1[1 + H:], bc_disp)

    gA = _field_sc(EA, PFA, t1, t2, fA0, fA1)
    fa = _field(gA, r2(ef_W1[0]), r2(ef_b1), ef_W2, r2(ef_b2), w3s, rrep, b3t)
    gB = _field_sc(EB, PFB, t1, t2, fB0, fB1)
    fb = _field(gB, r2(ef_W1[0]), r2(ef_b1), ef_W2, r2(ef_b2), w3s, rrep, b3t)
    f = jnp.concatenate([fa, fb], axis=0)

    fields = f.reshape(E, NPTS, 3)
    xi = jnp.broadcast_to(
        jnp.asarray(_XI, jnp.float32)[None, :, None], (E, NPTS, 1))
    return h, xi, fields, prop_I22


# speedup vs baseline: 6.5335x; 1.0008x over previous
import functools

import jax
import jax.numpy as jnp
from jax import lax
from jax.experimental import pallas as pl
from jax.experimental.pallas import tpu as pltpu
from jax.experimental.pallas import tpu_sc as plsc

N = 10000
E = 160000
H = 128
NODE_IN = 9
EDGE_DIM = 11
NPTS = 5

BN = 2000
BE = 8000

_XI = [0.0, 0.25, 0.5, 0.75, 1.0]

NC = 2
NS = 16
NPA = 10112
RPSA = NPA // NS

CCH = 40
PT0, PT1 = 5120, 4880
NCH0, NCH1 = PT0 // CCH, PT1 // CCH
C1OFF = NS * PT0

CHF = 40
EA, EB = 83200, 76800
PFA = (2880, 2320)
PFB = (2480, 2320)
BEF = 1600

_SC_MESH = plsc.VectorSubcoreMesh(core_axis_name="c", subcore_axis_name="s")

def _silu(v):
    return v * jax.nn.sigmoid(v)


def _conv_sc(a, b, c, i2a, i2b):
    @functools.partial(
        pl.kernel,
        out_type=jax.ShapeDtypeStruct((NC, NPA, H), jnp.float32),
        mesh=_SC_MESH,
        scratch_types=[
            pltpu.VMEM((1, CCH), jnp.int32),
            pltpu.VMEM((1, CCH), jnp.int32),
            pltpu.VMEM((CCH,), jnp.int32),
            pltpu.VMEM((CCH,), jnp.int32),
            pltpu.VMEM((CCH,), jnp.int32),
            pltpu.VMEM((CCH,), jnp.int32),
            pltpu.VMEM((CCH, H), jnp.float32),
            pltpu.VMEM((CCH, H), jnp.float32),
            pltpu.VMEM((CCH, H), jnp.float32),
            pltpu.VMEM((CCH, H), jnp.float32),
            pltpu.VMEM((CCH, H), jnp.float32),
            pltpu.VMEM((CCH, H), jnp.float32),
            pltpu.VMEM_SHARED((NPA, H), jnp.float32),
            pltpu.SemaphoreType.DMA,
            pltpu.SemaphoreType.DMA,
            pltpu.SemaphoreType.DMA,
            pltpu.SemaphoreType.DMA,
            pltpu.SemaphoreType.DMA,
            pltpu.SemaphoreType.DMA,
        ],
    )
    def k(a_hbm, b_hbm, c_hbm, i2a_hbm, i2b_hbm, out_hbm,
          ib0, ib1, is0, is1, id0, id1, ar0, ar1, br0, br1, cr0, cr1, acc,
          sa0, sa1, sb0, sb1, sc0, sc1):
        cid = lax.axis_index("c")
        sid = lax.axis_index("s")
        zrow = sid * RPSA

        @pl.loop(0, CCH)
        def _(r):
            @pl.loop(0, H, step=16)
            def _(cc):
                ar0[r, pl.ds(cc, 16)] = jnp.zeros((16,), jnp.float32)

        @pl.loop(0, RPSA - RPSA % CCH, step=CCH)
        def _(rr):
            pltpu.sync_copy(ar0, acc.at[pl.ds(zrow + rr, CCH)])
        pltpu.sync_copy(ar0.at[pl.ds(0, RPSA % CCH)],
                        acc.at[pl.ds(zrow + RPSA - RPSA % CCH, RPSA % CCH)])

        plsc.subcore_barrier()

        bufs = ((ib0, is0, id0, ar0, br0, cr0, sa0, sb0, sc0),
                (ib1, is1, id1, ar1, br1, cr1, sa1, sb1, sc1))

        def run(base, nch, i2_hbm):
            def issue(ci, ib, isb, idb, ar, br, cr, sa, sb, sc):
                off = base + ci * CCH
                pltpu.sync_copy(i2_hbm.at[sid, ci], ib)
                for o in (0, 16, CCH - 16):
                    v = ib[0, pl.ds(o, 16)]
                    isb[pl.ds(o, 16)] = jnp.bitwise_and(v, 65535)
                    idb[pl.ds(o, 16)] = jnp.right_shift(v, 16)
                pltpu.async_copy(a_hbm.at[isb], ar, sa)
                pltpu.async_copy(b_hbm.at[idb], br, sb)
                pltpu.async_copy(c_hbm.at[pl.ds(off, CCH)], cr, sc)

            issue(0, *bufs[0])
            issue(1, *bufs[1])

            @pl.loop(0, nch, step=2)
            def _(g):
                for p in range(2):
                    ib, isb, idb, ar, br, cr, sa, sb, sc = bufs[p]
                    ci = g + p
                    pltpu.make_async_copy(a_hbm.at[isb], ar, sa).wait()
                    pltpu.make_async_copy(b_hbm.at[idb], br, sb).wait()
                    pltpu.make_async_copy(c_hbm.at[pl.ds(base, CCH)], cr,
                                          sc).wait()

                    @pl.loop(0, CCH)
                    def _(r):
                        @pl.loop(0, H, step=16)
                        def _(cc):
                            sl = pl.ds(cc, 16)
                            v = ar[r, sl] + br[r, sl] + cr[r, sl]
                            ar[r, sl] = v / (1.0 + jnp.exp(-v))

                    pltpu.sync_copy(ar, acc.at[idb], add=True)

                    @pl.when(ci + 2 < nch)
                    def _():
                        issue(ci + 2, *bufs[p])

        @pl.when(cid == 0)
        def _():
            run(sid * PT0, NCH0, i2a_hbm)

        @pl.when(cid == 1)
        def _():
            run(C1OFF + sid * PT1, NCH1, i2b_hbm)

        plsc.subcore_barrier()
        pltpu.sync_copy(acc.at[pl.ds(zrow, RPSA)],
                        out_hbm.at[cid, pl.ds(zrow, RPSA)])

    return k(a, b, c, i2a, i2b)


def _field_sc(e_tot, pf, t1, t2, f2a, f2b):
    @functools.partial(
        pl.kernel,
        out_type=jax.ShapeDtypeStruct((e_tot, 128), jnp.float32),
        mesh=_SC_MESH,
        scratch_types=[
            pltpu.VMEM((1, CHF), jnp.int32),
            pltpu.VMEM((1, CHF), jnp.int32),
            pltpu.VMEM((CHF,), jnp.int32),
            pltpu.VMEM((CHF,), jnp.int32),
            pltpu.VMEM((CHF,), jnp.int32),
            pltpu.VMEM((CHF,), jnp.int32),
            pltpu.VMEM((CHF, 128), jnp.float32),
            pltpu.VMEM((CHF, 128), jnp.float32),
            pltpu.VMEM((CHF, 128), jnp.float32),
            pltpu.VMEM((CHF, 128), jnp.float32),
            pltpu.VMEM((CHF, 128), jnp.float32),
            pltpu.VMEM((CHF, 128), jnp.float32),
            pltpu.SemaphoreType.DMA,
            pltpu.SemaphoreType.DMA,
            pltpu.SemaphoreType.DMA,
            pltpu.SemaphoreType.DMA,
            pltpu.SemaphoreType.DMA,
            pltpu.SemaphoreType.DMA,
        ],
    )
    def k(t1_hbm, t2_hbm, f2a_hbm, f2b_hbm, out_hbm,
          ib0, ib1, is0, is1, id0, id1, g10, g11, g20, g21, ob0, ob1,
          s10, s11, s20, s21, so0, so1):
        cid = lax.axis_index("c")
        sid = lax.axis_index("s")

        bufs = ((ib0, is0, id0, g10, g20, ob0, s10, s20, so0),
                (ib1, is1, id1, g11, g21, ob1, s11, s21, so1))

        def run(base, nf, i2_hbm):
            def issue(ci, ib, isb, idb, g1, g2, ob, s1, s2, so):
                pltpu.sync_copy(i2_hbm.at[sid, ci], ib)
                for o in (0, 16, CHF - 16):
                    v = ib[0, pl.ds(o, 16)]
                    isb[pl.ds(o, 16)] = jnp.bitwise_and(v, 65535)
                    idb[pl.ds(o, 16)] = jnp.right_shift(v, 16)
                pltpu.async_copy(t1_hbm.at[isb], g1, s1)
                pltpu.async_copy(t2_hbm.at[idb], g2, s2)

            issue(0, *bufs[0])
            issue(1, *bufs[1])

            @pl.loop(0, nf, step=2)
            def _(g):
                for p in range(2):
                    ib, isb, idb, g1, g2, ob, s1, s2, so = bufs[p]
                    ci = g + p
                    pltpu.make_async_copy(t1_hbm.at[isb], g1, s1).wait()
                    pltpu.make_async_copy(t2_hbm.at[idb], g2, s2).wait()

                    @pl.when(ci >= 2)
                    def _():
                        pltpu.make_async_copy(
                            ob, out_hbm.at[pl.ds(base, CHF)], so).wait()

                    @pl.loop(0, CHF)
                    def _(r):
                        @pl.loop(0, 64, step=16)
                        def _(cc):
                            ob[r, pl.ds(cc, 16)] = (g1[r, pl.ds(cc, 16)]
                                                    + g2[r, pl.ds(cc, 16)])
                        ob[r, pl.ds(64, 16)] = (g1[r, pl.ds(64, 16)]
                                                * g2[r, pl.ds(64, 16)])

                    pltpu.async_copy(
                        ob, out_hbm.at[pl.ds(base + ci * CHF, CHF)], so)

                    @pl.when(ci + 2 < nf)
                    def _():
                        issue(ci + 2, *bufs[p])

            pltpu.make_async_copy(ob0, out_hbm.at[pl.ds(base, CHF)],
                                  so0).wait()
            pltpu.make_async_copy(ob1, out_hbm.at[pl.ds(base, CHF)],
                                  so1).wait()

        @pl.when(cid == 0)
        def _():
            run(sid * pf[0], pf[0] // CHF, f2a_hbm)

        @pl.when(cid == 1)
        def _():
            run(NS * pf[0] + sid * pf[1], pf[1] // CHF, f2b_hbm)

    return k(t1, t2, f2a, f2b)


def _enc_body(x_ref, w1_ref, b1_ref, w2_ref, b2_ref, ws_ref, wd_ref,
              h_ref, a_ref, b_ref):
    h1 = _silu(jnp.dot(x_ref[...], w1_ref[...],
                       preferred_element_type=jnp.float32) + b1_ref[...])
    h2 = _silu(jnp.dot(h1, w2_ref[...],
                       preferred_element_type=jnp.float32) + b2_ref[...])
    h_ref[...] = h2
    a_ref[...] = jnp.dot(h2, ws_ref[...], preferred_element_type=jnp.float32)
    b_ref[...] = jnp.dot(h2, wd_ref[...], preferred_element_type=jnp.float32)


def _encode(x, w1, b1, w2, b2, ws, wd):
    full = lambda i: (0, 0)
    return pl.pallas_call(
        _enc_body,
        grid=(N // BN,),
        in_specs=[
            pl.BlockSpec((BN, NODE_IN), lambda i: (i, 0)),
            pl.BlockSpec((NODE_IN, H), full),
            pl.BlockSpec((1, H), full),
            pl.BlockSpec((H, H), full),
            pl.BlockSpec((1, H), full),
            pl.BlockSpec((H, H), full),
            pl.BlockSpec((H, H), full),
        ],
        out_specs=[pl.BlockSpec((BN, H), lambda i: (i, 0))] * 3,
        out_shape=[jax.ShapeDtypeStruct((N, H), jnp.float32)] * 3,
    )(x, w1, b1, w2, b2, ws, wd)


def _edge_c_body(ea_ref, wa_ref, eb_ref, c_ref):
    c_ref[...] = (jnp.dot(ea_ref[...], wa_ref[...],
                          preferred_element_type=jnp.float32)
                  + eb_ref[...])


def _edge_c(ea, wa, eb):
    return pl.pallas_call(
        _edge_c_body,
        grid=(E // BE,),
        in_specs=[
            pl.BlockSpec((BE, EDGE_DIM), lambda i: (i, 0)),
            pl.BlockSpec((EDGE_DIM, H), lambda i: (0, 0)),
            pl.BlockSpec((1, H), lambda i: (0, 0)),
        ],
        out_specs=pl.BlockSpec((BE, H), lambda i: (i, 0)),
        out_shape=jax.ShapeDtypeStruct((E, H), jnp.float32),
    )(ea, wa, eb)


def _node_upd_body(h_ref, agg0_ref, agg1_ref, w1_ref, w2_ref, nb_ref,
                   ws_ref, wd_ref, h_ref_o, a_ref, b_ref):
    agg = agg0_ref[0] + agg1_ref[0]
    hn = _silu(jnp.dot(h_ref[...], w1_ref[...],
                       preferred_element_type=jnp.float32)
               + jnp.dot(agg, w2_ref[...],
                         preferred_element_type=jnp.float32)
               + nb_ref[...])
    h_ref_o[...] = hn
    a_ref[...] = jnp.dot(hn, ws_ref[...], preferred_element_type=jnp.float32)
    b_ref[...] = jnp.dot(hn, wd_ref[...], preferred_element_type=jnp.float32)


def _node_update(h, aggp, nw1, nw2, nb, ws, wd):
    full = lambda i: (0, 0)
    return pl.pallas_call(
        _node_upd_body,
        grid=(N // BN,),
        in_specs=[
            pl.BlockSpec((BN, H), lambda i: (i, 0)),
            pl.BlockSpec((1, BN, H), lambda i: (0, i, 0)),
            pl.BlockSpec((1, BN, H), lambda i: (1, i, 0)),
            pl.BlockSpec((H, H), full),
            pl.BlockSpec((H, H), full),
            pl.BlockSpec((1, H), full),
            pl.BlockSpec((H, H), full),
            pl.BlockSpec((H, H), full),
        ],
        out_specs=[pl.BlockSpec((BN, H), lambda i: (i, 0))] * 3,
        out_shape=[jax.ShapeDtypeStruct((N, H), jnp.float32)] * 3,
    )(h, aggp, aggp, nw1, nw2, nb, ws, wd)


def _node_final_body(h_ref, agg0_ref, agg1_ref, w1_ref, w2_ref, nb_ref,
                     wi_ref, wj_ref, bc_ref,
                     h_ref_o, t1_ref, t2_ref):
    agg = agg0_ref[0] + agg1_ref[0]
    hn = _silu(jnp.dot(h_ref[...], w1_ref[...],
                       preferred_element_type=jnp.float32)
               + jnp.dot(agg, w2_ref[...],
                         preferred_element_type=jnp.float32)
               + nb_ref[...])
    h_ref_o[...] = hn
    bc = bc_ref[...]
    xi = 0.25 * lax.broadcasted_iota(jnp.int32, (1, NPTS), 1).astype(
        jnp.float32)
    mi = 1.0 - bc * (1.0 - xi)
    mj = 1.0 - bc * xi
    ha = jnp.dot(hn, wi_ref[...], preferred_element_type=jnp.float32)
    hb = jnp.dot(hn, wj_ref[...], preferred_element_type=jnp.float32)
    zpad = jnp.zeros((hn.shape[0], 128 - 64 - NPTS), jnp.float32)
    t1_ref[...] = jnp.concatenate([ha, mi, zpad], axis=1)
    t2_ref[...] = jnp.concatenate([hb, mj, zpad], axis=1)


def _node_final(h, aggp, nw1, nw2, nb, wi, wj, bc):
    full = lambda i: (0, 0)
    return pl.pallas_call(
        _node_final_body,
        grid=(N // BN,),
        in_specs=[
            pl.BlockSpec((BN, H), lambda i: (i, 0)),
            pl.BlockSpec((1, BN, H), lambda i: (0, i, 0)),
            pl.BlockSpec((1, BN, H), lambda i: (1, i, 0)),
            pl.BlockSpec((H, H), full),
            pl.BlockSpec((H, H), full),
            pl.BlockSpec((1, H), full),
            pl.BlockSpec((H, 64), full),
            pl.BlockSpec((H, 64), full),
            pl.BlockSpec((BN, 1), lambda i: (i, 0)),
        ],
        out_specs=[
            pl.BlockSpec((BN, H), lambda i: (i, 0)),
            pl.BlockSpec((BN, 128), lambda i: (i, 0)),
            pl.BlockSpec((BN, 128), lambda i: (i, 0)),
        ],
        out_shape=[
            jax.ShapeDtypeStruct((N, H), jnp.float32),
            jax.ShapeDtypeStruct((N, 128), jnp.float32),
            jax.ShapeDtypeStruct((N, 128), jnp.float32),
        ],
    )(h, aggp, aggp, nw1, nw2, nb, wi, wj, bc)


def _field_body(g_ref, w0_ref, b1_ref, w2_ref, b2_ref, w3s_ref, r_ref,
                b3t_ref, f_ref):
    g = g_ref[...]
    base = g[:, :64]
    w0 = w0_ref[...]
    acc = None
    for k in range(NPTS):
        z = _silu(base + _XI[k] * w0 + b1_ref[...])
        z = _silu(jnp.dot(z, w2_ref[...],
                          preferred_element_type=jnp.float32) + b2_ref[...])
        t = jnp.dot(z, w3s_ref[64 * k:64 * (k + 1), :],
                    preferred_element_type=jnp.float32)
        acc = t if acc is None else acc + t
    mask15 = jnp.dot(g[:, 64:64 + NPTS], r_ref[...],
                     preferred_element_type=jnp.float32)
    f_ref[...] = (acc + b3t_ref[...]) * mask15


def _field(g, w0, b1, w2, b2, w3s, r, b3t):
    e_tot = g.shape[0]
    full = lambda i: (0, 0)
    return pl.pallas_call(
        _field_body,
        grid=(e_tot // BEF,),
        in_specs=[
            pl.BlockSpec((BEF, 128), lambda i: (i, 0)),
            pl.BlockSpec((1, 64), full),
            pl.BlockSpec((1, 64), full),
            pl.BlockSpec((64, 64), full),
            pl.BlockSpec((1, 64), full),
            pl.BlockSpec((NPTS * 64, NPTS * 3), full),
            pl.BlockSpec((NPTS, NPTS * 3), full),
            pl.BlockSpec((1, NPTS * 3), full),
        ],
        out_specs=pl.BlockSpec((BEF, NPTS * 3), lambda i: (i, 0)),
        out_shape=jax.ShapeDtypeStruct((e_tot, NPTS * 3), jnp.float32),
    )(g, w0, b1, w2, b2, w3s, r, b3t)


def kernel(x, edge_index, edge_attr, connectivity, bc_disp, prop_I22,
           enc_W1, enc_b1, enc_W2, enc_b2,
           conv0_eW, conv0_eb, conv0_nW, conv0_nb,
           conv1_eW, conv1_eb, conv1_nW, conv1_nb,
           conv2_eW, conv2_eb, conv2_nW, conv2_nb,
           ef_W1, ef_b1, ef_W2, ef_b2, ef_W3, ef_b3):
    convs = [(conv0_eW, conv0_eb, conv0_nW, conv0_nb),
             (conv1_eW, conv1_eb, conv1_nW, conv1_nb),
             (conv2_eW, conv2_eb, conv2_nW, conv2_nb)]

    r2 = lambda v: v.reshape(1, -1)

    def _split(svec, dvec, cut, pt0, pt1, ch):
        pk = svec + (dvec << 16)
        ia = pk[:cut].reshape(NS, pt0 // ch, 1, ch)
        ib_ = pk[cut:].reshape(NS, pt1 // ch, 1, ch)
        return ia, ib_

    i2a, i2b = _split(edge_index[0], edge_index[1], C1OFF, PT0, PT1, CCH)
    n1, n2 = connectivity[:, 0], connectivity[:, 1]

    w3s = jnp.zeros((NPTS * 64, NPTS * 3), jnp.float32)
    rrep = jnp.zeros((NPTS, NPTS * 3), jnp.float32)
    for k in range(NPTS):
        w3s = w3s.at[64 * k:64 * (k + 1), 3 * k:3 * k + 3].set(ef_W3)
        rrep = rrep.at[k, 3 * k:3 * k + 3].set(1.0)
    b3t = jnp.tile(ef_b3, NPTS)[None, :]

    h, a, b = _encode(x, enc_W1, r2(enc_b1), enc_W2, r2(enc_b2),
                      convs[0][0][:H], convs[0][0][H:2 * H])

    dep = (h[0, 0] * 0.0).astype(jnp.int32)
    n1d = n1 + dep
    fA0, fA1 = _split(n1d[:EA], n2[:EA], NS * PFA[0], PFA[0], PFA[1], CHF)
    fB0, fB1 = _split(n1d[EA:], n2[EA:], NS * PFB[0], PFB[0], PFB[1], CHF)

    for li in range(3):
        eW, eb, nW, nb = convs[li]
        c = _edge_c(edge_attr, eW[2 * H:], r2(eb))
        aggp = _conv_sc(a, b, c, i2a, i2b)
        if li < 2:
            nws, nwd = convs[li + 1][0][:H], convs[li + 1][0][H:2 * H]
            h, a, b = _node_update(h, aggp, nW[:H], nW[H:], r2(nb), nws, nwd)
        else:
            h, t1, t2 = _node_final(h, aggp, nW[:H], nW[H:], r2(nb),
                                    ef_W1[1:1 + H], ef_W1[1 + H:], bc_disp)

    gA = _field_sc(EA, PFA, t1, t2, fA0, fA1)
    fa = _field(gA, r2(ef_W1[0]), r2(ef_b1), ef_W2, r2(ef_b2), w3s, rrep, b3t)
    gB = _field_sc(EB, PFB, t1, t2, fB0, fB1)
    fb = _field(gB, r2(ef_W1[0]), r2(ef_b1), ef_W2, r2(ef_b2), w3s, rrep, b3t)
    f = jnp.concatenate([fa, fb], axis=0)

    fields = f.reshape(E, NPTS, 3)
    xi = jnp.broadcast_to(
        jnp.asarray(_XI, jnp.float32)[None, :, None], (E, NPTS, 1))
    return h, xi, fields, prop_I22
